# baseline (speedup 1.0000x reference)
import functools

import jax
import jax.numpy as jnp
from jax import lax
from jax.experimental import pallas as pl
from jax.experimental.pallas import tpu as pltpu
from jax.experimental.pallas import tpu_sc as plsc

N = 10000
D = 128
DH = 64
E = 320000
NPAD = 10240
NCORE = 2
NSUB = 16
RPT = NPAD // NSUB
CHUNK = 128
NCH = 160
NPADROWS = NPAD - N
NBUF = 2


def _sc_kernel_fn():
    mesh = plsc.VectorSubcoreMesh(core_axis_name="c", subcore_axis_name="s")

    @functools.partial(
        pl.kernel,
        mesh=mesh,
        compiler_params=pltpu.CompilerParams(use_tc_tiling_on_sc=False),
        out_type=(
            jax.ShapeDtypeStruct((NCORE, NPAD, DH), jnp.float32),
            jax.ShapeDtypeStruct((NCORE, NPAD, DH), jnp.float32),
            jax.ShapeDtypeStruct((NPAD,), jnp.float32),
        ),
        scratch_types=[
            pltpu.VMEM((NCH, CHUNK), jnp.int32),
            pltpu.VMEM((NCH, CHUNK), jnp.int32),
            pltpu.VMEM((CHUNK, DH), jnp.float32),
            pltpu.VMEM((CHUNK, DH), jnp.float32),
            pltpu.VMEM((64, D), jnp.float32),
            pltpu.VMEM((64, DH), jnp.float32),
            pltpu.VMEM((64, DH), jnp.float32),
            pltpu.VMEM((CHUNK,), jnp.float32),
            pltpu.VMEM((RPT,), jnp.float32),
            pltpu.VMEM_SHARED((NPAD,), jnp.float32),
            pltpu.VMEM_SHARED((NPAD, DH), jnp.float32),
            pltpu.SemaphoreType.DMA,
            pltpu.SemaphoreType.DMA,
            pltpu.SemaphoreType.DMA,
            pltpu.SemaphoreType.DMA,
            pltpu.SemaphoreType.DMA,
        ],
    )
    def sc_kernel(x_hbm, srcT_hbm, dstT_hbm,
                  s_out, xs_out, dinv_out,
                  src_v, dst_v, rb0, rb1, xbuf_v, xh0_v, xh1_v,
                  ones_v, dloc_v, deg_sh, acc_sh,
                  dsem, gs0, gs1, ss0, ss1):
        c = lax.axis_index("c")
        s = lax.axis_index("s")
        row0 = s * RPT
        rbufs = [rb0, rb1]
        gsems = [gs0, gs1]
        ssems = [ss0, ss1]

        for j in range(8):
            ones_v[pl.ds(j * 16, 16)] = jnp.ones((16,), jnp.float32)

        def _zero_dloc(k, _):
            dloc_v[pl.ds(k * 16, 16)] = jnp.zeros((16,), jnp.float32)
            return 0
        lax.fori_loop(0, RPT // 16, _zero_dloc, 0)
        pltpu.sync_copy(dloc_v, deg_sh.at[pl.ds(row0, RPT)])
        plsc.subcore_barrier()

        _scB = jax.named_scope("phaseB_deg"); _scB.__enter__()
        pltpu.sync_copy(dstT_hbm.at[pl.ds(s * NCH, NCH)], dst_v)

        def _deg(i, _):
            pltpu.async_copy(ones_v, deg_sh.at[dst_v.at[i]], dsem, add=True)

            @pl.when(i >= 8)
            def _():
                pltpu.make_async_copy(
                    ones_v, deg_sh.at[dst_v.at[i - 8]], dsem).wait()
            return 0
        lax.fori_loop(0, NCH, _deg, 0)
        for i in range(NCH - 8, NCH):
            pltpu.make_async_copy(
                ones_v, deg_sh.at[dst_v.at[i]], dsem).wait()
        plsc.subcore_barrier()
        _scB.__exit__(None, None, None)

        pltpu.sync_copy(deg_sh.at[pl.ds(row0, RPT)], dloc_v)

        def _dinv(k, _):
            dv = dloc_v[pl.ds(k * 16, 16)] + 1.0
            bits = lax.bitcast_convert_type(dv, jnp.int32)
            y = lax.bitcast_convert_type(
                jnp.int32(0x5F3759DF) - (bits >> 1), jnp.float32)
            half = dv * 0.5
            y = y * (1.5 - half * y * y)
            y = y * (1.5 - half * y * y)
            y = y * (1.5 - half * y * y)
            dloc_v[pl.ds(k * 16, 16)] = y
            return 0
        lax.fori_loop(0, RPT // 16, _dinv, 0)
        pltpu.sync_copy(dloc_v, dinv_out.at[pl.ds(row0, RPT)])

        _scD = jax.named_scope("phaseD_scale"); _scD.__enter__()
        def _scale(ch, _):
            r0 = row0 + ch * 64
            pltpu.sync_copy(x_hbm.at[pl.ds(r0, 64)], xbuf_v)

            def _grp(g, _):
                dv16 = dloc_v[pl.ds(ch * 64 + g * 16, 16)]
                for rr in range(16):
                    d = dv16[rr]
                    row = g * 16 + rr
                    for j in range(4):
                        sl = pl.ds(j * 16, 16)
                        xh0_v[row, sl] = xbuf_v[row, sl] * d
                    for j in range(4):
                        sl = pl.ds(j * 16, 16)
                        xh1_v[row, sl] = xbuf_v[row, pl.ds(64 + j * 16, 16)] * d
                return 0
            lax.fori_loop(0, 4, _grp, 0)
            pltpu.sync_copy(xh0_v, xs_out.at[0, pl.ds(r0, 64)])
            pltpu.sync_copy(xh1_v, xs_out.at[1, pl.ds(r0, 64)])

            @pl.when(c == 0)
            def _():
                pltpu.sync_copy(xh0_v, acc_sh.at[pl.ds(r0, 64)])

            @pl.when(c == 1)
            def _():
                pltpu.sync_copy(xh1_v, acc_sh.at[pl.ds(r0, 64)])
            return 0
        lax.fori_loop(0, RPT // 64, _scale, 0)
        plsc.subcore_barrier()
        _scD.__exit__(None, None, None)
        _scE = jax.named_scope("phaseE_main"); _scE.__enter__()

        pltpu.sync_copy(srcT_hbm.at[pl.ds(s * NCH, NCH)], src_v)

        def _gstart(k, i):
            pltpu.async_copy(
                xs_out.at[c].at[src_v.at[i]], rbufs[k], gsems[k])

        def _gwait(k, i):
            pltpu.make_async_copy(
                xs_out.at[c].at[src_v.at[i]], rbufs[k], gsems[k]).wait()

        def _sstart(k, i):
            pltpu.async_copy(
                rbufs[k], acc_sh.at[dst_v.at[i]], ssems[k], add=True)

        def _swait(k, i):
            pltpu.make_async_copy(
                rbufs[k], acc_sh.at[dst_v.at[i]], ssems[k]).wait()

        for k in range(NBUF):
            _gstart(k, k)

        def _quad(p, _):
            i0 = NBUF * p
            for k in range(NBUF):
                i = i0 + k
                _gwait(k, i)
                _sstart(k, i)
                km1 = (k - 1) % NBUF
                if km1 == NBUF - 1:
                    @pl.when(p > 0)
                    def _():
                        _swait(km1, i - 1)
                        _gstart(km1, i + NBUF - 1)
                else:
                    @pl.when(i + NBUF - 1 < NCH)
                    def _():
                        _swait(km1, i - 1)
                        _gstart(km1, i + NBUF - 1)
            return 0
        lax.fori_loop(0, NCH // NBUF, _quad, 0)
        for k in range(NBUF - 1):
            _swait(k, NCH - NBUF + k)
        _swait(NBUF - 1, NCH - 1)
        plsc.subcore_barrier()
        _scE.__exit__(None, None, None)

        def _out(k, _):
            r0 = row0 + k * CHUNK
            pltpu.sync_copy(acc_sh.at[pl.ds(r0, CHUNK)], rb0)
            pltpu.sync_copy(rb0, s_out.at[c, pl.ds(r0, CHUNK)])
            return 0
        lax.fori_loop(0, RPT // CHUNK, _out, 0)

    return sc_kernel


_SC_KERNEL = _sc_kernel_fn()

_TC_ROWS = 2000


def _tc_body(s_ref, dinv_ref, w_ref, b_ref, o_ref):
    u = jnp.concatenate([s_ref[0], s_ref[1]], axis=1)
    acc = jnp.dot(u, w_ref[...], preferred_element_type=jnp.float32)
    o_ref[...] = acc * dinv_ref[...] + b_ref[...]


def kernel(x, edge_index, W, b):
    src = edge_index[0].astype(jnp.int32)
    dst = edge_index[1].astype(jnp.int32)
    ept_real = E // NSUB
    npad_e = NCH * CHUNK - ept_real
    pad = N + jnp.arange(npad_e, dtype=jnp.int32) % NPADROWS
    pad = jnp.broadcast_to(pad, (NSUB, npad_e))
    srcT = jnp.concatenate([src.reshape(NSUB, ept_real), pad], axis=1)
    dstT = jnp.concatenate([dst.reshape(NSUB, ept_real), pad], axis=1)
    srcT = srcT.reshape(NSUB * NCH, CHUNK)
    dstT = dstT.reshape(NSUB * NCH, CHUNK)
    x_pad = jnp.concatenate(
        [x, jnp.zeros((NPAD - N, D), jnp.float32)], axis=0)

    S, _, dinv = _SC_KERNEL(x_pad, srcT, dstT)

    out = pl.pallas_call(
        _tc_body,
        grid=(N // _TC_ROWS,),
        in_specs=[
            pl.BlockSpec((NCORE, _TC_ROWS, DH), lambda i: (0, i, 0)),
            pl.BlockSpec((_TC_ROWS, 1), lambda i: (i, 0)),
            pl.BlockSpec((D, D), lambda i: (0, 0)),
            pl.BlockSpec((1, D), lambda i: (0, 0)),
        ],
        out_specs=pl.BlockSpec((_TC_ROWS, D), lambda i: (i, 0)),
        out_shape=jax.ShapeDtypeStruct((N, D), jnp.float32),
    )(S, dinv.reshape(NPAD, 1)[:N], W, b.reshape(1, D))

    return out


# speedup vs baseline: 1.2524x; 1.2524x over previous
import functools

import jax
import jax.numpy as jnp
from jax import lax
from jax.experimental import pallas as pl
from jax.experimental.pallas import tpu as pltpu
from jax.experimental.pallas import tpu_sc as plsc

N = 10000
D = 128
DH = 64
E = 320000
NPAD = 10240
NCORE = 2
NSUB = 16
RPT = NPAD // NSUB
CHUNK = 128
NCH = 160
NPADROWS = NPAD - N
NBUF = 4
QCH = 40


def _sc_kernel_fn():
    mesh = plsc.VectorSubcoreMesh(core_axis_name="c", subcore_axis_name="s")

    @functools.partial(
        pl.kernel,
        mesh=mesh,
        compiler_params=pltpu.CompilerParams(
            use_tc_tiling_on_sc=False, needs_layout_passes=False),
        out_type=(
            jax.ShapeDtypeStruct((NCORE, NPAD, DH), jnp.float32),
            jax.ShapeDtypeStruct((NCORE, NPAD, DH), jnp.float32),
            jax.ShapeDtypeStruct((NPAD,), jnp.float32),
        ),
        scratch_types=[
            pltpu.VMEM((QCH, CHUNK), jnp.int32),
            pltpu.VMEM((QCH, CHUNK), jnp.int32),
            pltpu.VMEM((QCH, CHUNK), jnp.int32),
            pltpu.VMEM((QCH, CHUNK), jnp.int32),
            pltpu.VMEM((CHUNK, DH), jnp.float32),
            pltpu.VMEM((CHUNK, DH), jnp.float32),
            pltpu.VMEM((CHUNK, DH), jnp.float32),
            pltpu.VMEM((CHUNK, DH), jnp.float32),
            pltpu.VMEM((64, D), jnp.float32),
            pltpu.VMEM((64, DH), jnp.float32),
            pltpu.VMEM((64, DH), jnp.float32),
            pltpu.VMEM((CHUNK,), jnp.float32),
            pltpu.VMEM((RPT,), jnp.float32),
            pltpu.VMEM_SHARED((NPAD,), jnp.float32),
            pltpu.VMEM_SHARED((NPAD, DH), jnp.float32),
            pltpu.SemaphoreType.DMA,
            pltpu.SemaphoreType.DMA,
            pltpu.SemaphoreType.DMA,
            pltpu.SemaphoreType.DMA,
            pltpu.SemaphoreType.DMA,
            pltpu.SemaphoreType.DMA,
            pltpu.SemaphoreType.DMA,
            pltpu.SemaphoreType.DMA,
            pltpu.SemaphoreType.DMA,
        ],
    )
    def sc_kernel(x_hbm, srcT_hbm, dstT_hbm,
                  s_out, xs_out, dinv_out,
                  sq0, sq1, dq0, dq1, rb0, rb1, rb2, rb3,
                  xbuf_v, xh0_v, xh1_v,
                  ones_v, dloc_v, deg_sh, acc_sh,
                  dsem, gs0, gs1, gs2, gs3, ss0, ss1, ss2, ss3):
        c = lax.axis_index("c")
        s = lax.axis_index("s")
        row0 = s * RPT
        rbufs = [rb0, rb1, rb2, rb3]
        gsems = [gs0, gs1, gs2, gs3]
        ssems = [ss0, ss1, ss2, ss3]
        sqs = [sq0, sq1]
        dqs = [dq0, dq1]

        for j in range(8):
            ones_v[pl.ds(j * 16, 16)] = jnp.ones((16,), jnp.float32)

        def _zero_dloc(k, _):
            dloc_v[pl.ds(k * 16, 16)] = jnp.zeros((16,), jnp.float32)
            return 0
        lax.fori_loop(0, RPT // 16, _zero_dloc, 0)
        pltpu.sync_copy(dloc_v, deg_sh.at[pl.ds(row0, RPT)])
        plsc.subcore_barrier()

        _scB = jax.named_scope("phaseB_deg"); _scB.__enter__()
        for q in range(NCH // QCH):
            dq = dqs[q % 2]
            pltpu.sync_copy(
                dstT_hbm.at[pl.ds(s * NCH + q * QCH, QCH)], dq)

            def _deg(i, _, dq=dq):
                pltpu.async_copy(ones_v, deg_sh.at[dq.at[i]], dsem, add=True)

                @pl.when(i >= 8)
                def _():
                    pltpu.make_async_copy(
                        ones_v, deg_sh.at[dq.at[i - 8]], dsem).wait()
                return 0
            lax.fori_loop(0, QCH, _deg, 0)
            for i in range(QCH - 8, QCH):
                pltpu.make_async_copy(
                    ones_v, deg_sh.at[dq.at[i]], dsem).wait()
        plsc.subcore_barrier()
        _scB.__exit__(None, None, None)

        pltpu.sync_copy(deg_sh.at[pl.ds(row0, RPT)], dloc_v)

        def _dinv(k, _):
            dv = dloc_v[pl.ds(k * 16, 16)] + 1.0
            bits = lax.bitcast_convert_type(dv, jnp.int32)
            y = lax.bitcast_convert_type(
                jnp.int32(0x5F3759DF) - (bits >> 1), jnp.float32)
            half = dv * 0.5
            y = y * (1.5 - half * y * y)
            y = y * (1.5 - half * y * y)
            y = y * (1.5 - half * y * y)
            dloc_v[pl.ds(k * 16, 16)] = y
            return 0
        lax.fori_loop(0, RPT // 16, _dinv, 0)
        pltpu.sync_copy(dloc_v, dinv_out.at[pl.ds(row0, RPT)])

        _scD = jax.named_scope("phaseD_scale"); _scD.__enter__()
        def _scale(ch, _):
            r0 = row0 + ch * 64
            pltpu.sync_copy(x_hbm.at[pl.ds(r0, 64)], xbuf_v)

            def _row(rr, _):
                idxv = jnp.zeros((16,), jnp.int32) + (ch * 64 + rr)
                dvb = plsc.load_gather(dloc_v, [idxv])
                for j in range(4):
                    sl = pl.ds(j * 16, 16)
                    xh0_v[rr, sl] = xbuf_v[rr, sl] * dvb
                for j in range(4):
                    sl = pl.ds(j * 16, 16)
                    xh1_v[rr, sl] = xbuf_v[rr, pl.ds(64 + j * 16, 16)] * dvb
                return 0
            lax.fori_loop(0, 64, _row, 0)
            pltpu.sync_copy(xh0_v, xs_out.at[0, pl.ds(r0, 64)])
            pltpu.sync_copy(xh1_v, xs_out.at[1, pl.ds(r0, 64)])

            @pl.when(c == 0)
            def _():
                pltpu.sync_copy(xh0_v, acc_sh.at[pl.ds(r0, 64)])

            @pl.when(c == 1)
            def _():
                pltpu.sync_copy(xh1_v, acc_sh.at[pl.ds(r0, 64)])
            return 0
        lax.fori_loop(0, RPT // 64, _scale, 0)
        plsc.subcore_barrier()
        _scD.__exit__(None, None, None)
        _scE = jax.named_scope("phaseE_main"); _scE.__enter__()

        def _gstart(k, sq, i):
            pltpu.async_copy(
                xs_out.at[c].at[sq.at[i]], rbufs[k], gsems[k])

        def _gwait(k, sq, i):
            pltpu.make_async_copy(
                xs_out.at[c].at[sq.at[i]], rbufs[k], gsems[k]).wait()

        def _sstart(k, dq, i):
            pltpu.async_copy(
                rbufs[k], acc_sh.at[dq.at[i]], ssems[k], add=True)

        def _swait(k, dq, i):
            pltpu.make_async_copy(
                rbufs[k], acc_sh.at[dq.at[i]], ssems[k]).wait()

        for q in range(NCH // QCH):
            sq = sqs[q % 2]
            dq = dqs[q % 2]
            pltpu.sync_copy(
                srcT_hbm.at[pl.ds(s * NCH + q * QCH, QCH)], sq)
            pltpu.sync_copy(
                dstT_hbm.at[pl.ds(s * NCH + q * QCH, QCH)], dq)
            for k in range(NBUF):
                _gstart(k, sq, k)

            def _quad(p, _, sq=sq, dq=dq):
                i0 = NBUF * p
                for k in range(NBUF):
                    i = i0 + k
                    _gwait(k, sq, i)
                    _sstart(k, dq, i)
                    km1 = (k - 1) % NBUF
                    if km1 == NBUF - 1:
                        @pl.when(p > 0)
                        def _():
                            _swait(km1, dq, i - 1)
                            _gstart(km1, sq, i + NBUF - 1)
                    else:
                        @pl.when(i + NBUF - 1 < QCH)
                        def _():
                            _swait(km1, dq, i - 1)
                            _gstart(km1, sq, i + NBUF - 1)
                return 0
            lax.fori_loop(0, QCH // NBUF, _quad, 0)
            for k in range(NBUF - 1):
                _swait(k, dq, QCH - NBUF + k)
            _swait(NBUF - 1, dq, QCH - 1)
        plsc.subcore_barrier()
        _scE.__exit__(None, None, None)

        def _out(k, _):
            r0 = row0 + k * CHUNK
            pltpu.sync_copy(acc_sh.at[pl.ds(r0, CHUNK)], rb0)
            pltpu.sync_copy(rb0, s_out.at[c, pl.ds(r0, CHUNK)])
            return 0
        lax.fori_loop(0, RPT // CHUNK, _out, 0)

    return sc_kernel


_SC_KERNEL = _sc_kernel_fn()

_TC_ROWS = 2000


def _tc_body(s_ref, dinv_ref, w_ref, b_ref, o_ref):
    u = jnp.concatenate([s_ref[0], s_ref[1]], axis=1)
    acc = jnp.dot(u, w_ref[...], preferred_element_type=jnp.float32)
    o_ref[...] = acc * dinv_ref[...] + b_ref[...]


def kernel(x, edge_index, W, b):
    src = edge_index[0].astype(jnp.int32)
    dst = edge_index[1].astype(jnp.int32)
    ept_real = E // NSUB
    npad_e = NCH * CHUNK - ept_real
    pad = N + jnp.arange(npad_e, dtype=jnp.int32) % NPADROWS
    pad = jnp.broadcast_to(pad, (NSUB, npad_e))
    srcT = jnp.concatenate([src.reshape(NSUB, ept_real), pad], axis=1)
    dstT = jnp.concatenate([dst.reshape(NSUB, ept_real), pad], axis=1)
    srcT = srcT.reshape(NSUB * NCH, CHUNK)
    dstT = dstT.reshape(NSUB * NCH, CHUNK)
    x_pad = jnp.concatenate(
        [x, jnp.zeros((NPAD - N, D), jnp.float32)], axis=0)

    S, _, dinv = _SC_KERNEL(x_pad, srcT, dstT)

    out = pl.pallas_call(
        _tc_body,
        grid=(N // _TC_ROWS,),
        in_specs=[
            pl.BlockSpec((NCORE, _TC_ROWS, DH), lambda i: (0, i, 0)),
            pl.BlockSpec((_TC_ROWS, 1), lambda i: (i, 0)),
            pl.BlockSpec((D, D), lambda i: (0, 0)),
            pl.BlockSpec((1, D), lambda i: (0, 0)),
        ],
        out_specs=pl.BlockSpec((_TC_ROWS, D), lambda i: (i, 0)),
        out_shape=jax.ShapeDtypeStruct((N, D), jnp.float32),
    )(S, dinv.reshape(NPAD, 1)[:N], W, b.reshape(1, D))

    return out


# speedup vs baseline: 1.3771x; 1.0996x over previous
import functools

import jax
import jax.numpy as jnp
from jax import lax
from jax.experimental import pallas as pl
from jax.experimental.pallas import tpu as pltpu
from jax.experimental.pallas import tpu_sc as plsc

N = 10000
D = 128
DH = 64
E = 320000
NPAD = 10240
NCORE = 2
NSUB = 16
RPT = NPAD // NSUB
CHUNK = 128
NCH = 160
NPADROWS = NPAD - N
NBUF = 5
QCH = 40
DCH = RPT // 64


def _sc_kernel_fn():
    mesh = plsc.VectorSubcoreMesh(core_axis_name="c", subcore_axis_name="s")

    @functools.partial(
        pl.kernel,
        mesh=mesh,
        compiler_params=pltpu.CompilerParams(
            use_tc_tiling_on_sc=False, needs_layout_passes=False),
        out_type=(
            jax.ShapeDtypeStruct((NCORE, NPAD, DH), jnp.float32),
            jax.ShapeDtypeStruct((NCORE, NPAD, DH), jnp.float32),
            jax.ShapeDtypeStruct((NPAD,), jnp.float32),
        ),
        scratch_types=[
            pltpu.VMEM((QCH, CHUNK), jnp.int32),
            pltpu.VMEM((QCH, CHUNK), jnp.int32),
            pltpu.VMEM((QCH, CHUNK), jnp.int32),
            pltpu.VMEM((QCH, CHUNK), jnp.int32),
            pltpu.VMEM((CHUNK, DH), jnp.float32),
            pltpu.VMEM((CHUNK, DH), jnp.float32),
            pltpu.VMEM((CHUNK, DH), jnp.float32),
            pltpu.VMEM((CHUNK, DH), jnp.float32),
            pltpu.VMEM((CHUNK, DH), jnp.float32),
            pltpu.VMEM((64, D), jnp.float32),
            pltpu.VMEM((64, D), jnp.float32),
            pltpu.VMEM((CHUNK,), jnp.float32),
            pltpu.VMEM((RPT,), jnp.float32),
            pltpu.VMEM_SHARED((NPAD,), jnp.float32),
            pltpu.VMEM_SHARED((NPAD, DH), jnp.float32),
            pltpu.SemaphoreType.DMA,
            pltpu.SemaphoreType.DMA,
            pltpu.SemaphoreType.DMA,
            pltpu.SemaphoreType.DMA,
            pltpu.SemaphoreType.DMA,
            pltpu.SemaphoreType.DMA,
            pltpu.SemaphoreType.DMA,
            pltpu.SemaphoreType.DMA,
            pltpu.SemaphoreType.DMA,
            pltpu.SemaphoreType.DMA,
            pltpu.SemaphoreType.DMA,
        ],
    )
    def sc_kernel(x_hbm, srcT_hbm, dstT_hbm,
                  s_out, xs_out, dinv_out,
                  sq0, sq1, dq0, dq1, rb0, rb1, rb2, rb3, rb4,
                  xb0, xb1, ones_v, dloc_v, deg_sh, acc_sh,
                  dsem, gs0, gs1, gs2, gs3, gs4, ss0, ss1, ss2, ss3, ss4):
        c = lax.axis_index("c")
        s = lax.axis_index("s")
        row0 = s * RPT
        rbufs = [rb0, rb1, rb2, rb3, rb4]
        gsems = [gs0, gs1, gs2, gs3, gs4]
        ssems = [ss0, ss1, ss2, ss3, ss4]
        sqs = [sq0, sq1]
        dqs = [dq0, dq1]
        xbufs = [xb0, xb1]

        for j in range(8):
            ones_v[pl.ds(j * 16, 16)] = jnp.ones((16,), jnp.float32)

        def _zero_dloc(k, _):
            dloc_v[pl.ds(k * 16, 16)] = jnp.zeros((16,), jnp.float32)
            return 0
        lax.fori_loop(0, RPT // 16, _zero_dloc, 0)
        pltpu.sync_copy(dloc_v, deg_sh.at[pl.ds(row0, RPT)])
        plsc.subcore_barrier()

        _scB = jax.named_scope("phaseB_deg"); _scB.__enter__()
        for q in range(NCH // QCH):
            dq = dqs[q % 2]
            pltpu.sync_copy(
                dstT_hbm.at[pl.ds(s * NCH + q * QCH, QCH)], dq)

            def _deg(i, _, dq=dq):
                pltpu.async_copy(ones_v, deg_sh.at[dq.at[i]], dsem, add=True)

                @pl.when(i >= 8)
                def _():
                    pltpu.make_async_copy(
                        ones_v, deg_sh.at[dq.at[i - 8]], dsem).wait()
                return 0
            lax.fori_loop(0, QCH, _deg, 0)
            for i in range(QCH - 8, QCH):
                pltpu.make_async_copy(
                    ones_v, deg_sh.at[dq.at[i]], dsem).wait()
        plsc.subcore_barrier()
        _scB.__exit__(None, None, None)

        pltpu.sync_copy(deg_sh.at[pl.ds(row0, RPT)], dloc_v)

        def _dinv(k, _):
            dv = dloc_v[pl.ds(k * 16, 16)] + 1.0
            bits = lax.bitcast_convert_type(dv, jnp.int32)
            y = lax.bitcast_convert_type(
                jnp.int32(0x5F3759DF) - (bits >> 1), jnp.float32)
            half = dv * 0.5
            y = y * (1.5 - half * y * y)
            y = y * (1.5 - half * y * y)
            y = y * (1.5 - half * y * y)
            dloc_v[pl.ds(k * 16, 16)] = y
            return 0
        lax.fori_loop(0, RPT // 16, _dinv, 0)
        pltpu.sync_copy(dloc_v, dinv_out.at[pl.ds(row0, RPT)])

        _scD = jax.named_scope("phaseD_scale"); _scD.__enter__()
        in_sems = [gs0, gs1]
        o0_sems = [gs2, gs3]
        o1_sems = [ss0, ss1]
        acc_sems = [ss2, ss3]

        def _d_bufs(ch):
            p = ch % 2
            return rbufs[0].at[pl.ds(64 * p, 64)], rbufs[1].at[pl.ds(64 * p, 64)]

        def _d_in_start(ch):
            pltpu.async_copy(
                x_hbm.at[pl.ds(row0 + ch * 64, 64)],
                xbufs[ch % 2], in_sems[ch % 2])

        def _d_in_wait(ch):
            pltpu.make_async_copy(
                x_hbm.at[pl.ds(row0 + ch * 64, 64)],
                xbufs[ch % 2], in_sems[ch % 2]).wait()

        def _d_out_start(ch):
            r0 = row0 + ch * 64
            p = ch % 2
            oh0, oh1 = _d_bufs(ch)
            pltpu.async_copy(oh0, xs_out.at[0, pl.ds(r0, 64)], o0_sems[p])
            pltpu.async_copy(oh1, xs_out.at[1, pl.ds(r0, 64)], o1_sems[p])

            @pl.when(c == 0)
            def _():
                pltpu.async_copy(
                    oh0, acc_sh.at[pl.ds(r0, 64)], acc_sems[p])

            @pl.when(c == 1)
            def _():
                pltpu.async_copy(
                    oh1, acc_sh.at[pl.ds(r0, 64)], acc_sems[p])

        def _d_out_wait(ch):
            r0 = row0 + ch * 64
            p = ch % 2
            oh0, oh1 = _d_bufs(ch)
            pltpu.make_async_copy(
                oh0, xs_out.at[0, pl.ds(r0, 64)], o0_sems[p]).wait()
            pltpu.make_async_copy(
                oh1, xs_out.at[1, pl.ds(r0, 64)], o1_sems[p]).wait()

            @pl.when(c == 0)
            def _():
                pltpu.make_async_copy(
                    oh0, acc_sh.at[pl.ds(r0, 64)], acc_sems[p]).wait()

            @pl.when(c == 1)
            def _():
                pltpu.make_async_copy(
                    oh1, acc_sh.at[pl.ds(r0, 64)], acc_sems[p]).wait()

        _d_in_start(0)
        for ch in range(DCH):
            if ch + 1 < DCH:
                _d_in_start(ch + 1)
            _d_in_wait(ch)
            if ch >= 2:
                _d_out_wait(ch - 2)
            p = ch % 2
            xin = xbufs[p]
            ob0, ob1 = rbufs[0], rbufs[1]

            def _row(rr, _, ch=ch, xin=xin, p=p, ob0=ob0, ob1=ob1):
                idxv = jnp.zeros((16,), jnp.int32) + (ch * 64 + rr)
                dvb = plsc.load_gather(dloc_v, [idxv])
                for j in range(4):
                    sl = pl.ds(j * 16, 16)
                    ob0[64 * p + rr, sl] = xin[rr, sl] * dvb
                for j in range(4):
                    sl = pl.ds(j * 16, 16)
                    ob1[64 * p + rr, sl] = xin[rr, pl.ds(64 + j * 16, 16)] * dvb
                return 0
            lax.fori_loop(0, 64, _row, 0)
            _d_out_start(ch)
        _d_out_wait(DCH - 2)
        _d_out_wait(DCH - 1)
        plsc.subcore_barrier()
        _scD.__exit__(None, None, None)
        _scE = jax.named_scope("phaseE_main"); _scE.__enter__()

        def _gstart(k, sq, i):
            pltpu.async_copy(
                xs_out.at[c].at[sq.at[i]], rbufs[k], gsems[k])

        def _gwait(k, sq, i):
            pltpu.make_async_copy(
                xs_out.at[c].at[sq.at[i]], rbufs[k], gsems[k]).wait()

        def _sstart(k, dq, i):
            pltpu.async_copy(
                rbufs[k], acc_sh.at[dq.at[i]], ssems[k], add=True)

        def _swait(k, dq, i):
            pltpu.make_async_copy(
                rbufs[k], acc_sh.at[dq.at[i]], ssems[k]).wait()

        for q in range(NCH // QCH):
            sq = sqs[q % 2]
            dq = dqs[q % 2]
            pltpu.sync_copy(
                srcT_hbm.at[pl.ds(s * NCH + q * QCH, QCH)], sq)
            pltpu.sync_copy(
                dstT_hbm.at[pl.ds(s * NCH + q * QCH, QCH)], dq)
            for k in range(NBUF):
                _gstart(k, sq, k)

            def _quad(p, _, sq=sq, dq=dq):
                i0 = NBUF * p
                for k in range(NBUF):
                    i = i0 + k
                    _gwait(k, sq, i)
                    _sstart(k, dq, i)
                    km1 = (k - 1) % NBUF
                    if km1 == NBUF - 1:
                        @pl.when(p > 0)
                        def _():
                            _swait(km1, dq, i - 1)
                            _gstart(km1, sq, i + NBUF - 1)
                    else:
                        @pl.when(i + NBUF - 1 < QCH)
                        def _():
                            _swait(km1, dq, i - 1)
                            _gstart(km1, sq, i + NBUF - 1)
                return 0
            lax.fori_loop(0, QCH // NBUF, _quad, 0)
            for k in range(NBUF - 1):
                _swait(k, dq, QCH - NBUF + k)
            _swait(NBUF - 1, dq, QCH - 1)
        plsc.subcore_barrier()
        _scE.__exit__(None, None, None)

        def _out(k, _):
            r0 = row0 + k * CHUNK
            pltpu.sync_copy(acc_sh.at[pl.ds(r0, CHUNK)], rb0)
            pltpu.sync_copy(rb0, s_out.at[c, pl.ds(r0, CHUNK)])
            return 0
        lax.fori_loop(0, RPT // CHUNK, _out, 0)

    return sc_kernel


_SC_KERNEL = _sc_kernel_fn()

_TC_ROWS = 2000


def _tc_body(s_ref, dinv_ref, w_ref, b_ref, o_ref):
    u = jnp.concatenate([s_ref[0], s_ref[1]], axis=1)
    acc = jnp.dot(u, w_ref[...], preferred_element_type=jnp.float32)
    o_ref[...] = acc * dinv_ref[...] + b_ref[...]


def kernel(x, edge_index, W, b):
    src = edge_index[0].astype(jnp.int32)
    dst = edge_index[1].astype(jnp.int32)
    ept_real = E // NSUB
    npad_e = NCH * CHUNK - ept_real
    pad = N + jnp.arange(npad_e, dtype=jnp.int32) % NPADROWS
    pad = jnp.broadcast_to(pad, (NSUB, npad_e))
    srcT = jnp.concatenate([src.reshape(NSUB, ept_real), pad], axis=1)
    dstT = jnp.concatenate([dst.reshape(NSUB, ept_real), pad], axis=1)
    srcT = srcT.reshape(NSUB * NCH, CHUNK)
    dstT = dstT.reshape(NSUB * NCH, CHUNK)
    x_pad = jnp.concatenate(
        [x, jnp.zeros((NPAD - N, D), jnp.float32)], axis=0)

    S, _, dinv = _SC_KERNEL(x_pad, srcT, dstT)

    out = pl.pallas_call(
        _tc_body,
        grid=(N // _TC_ROWS,),
        in_specs=[
            pl.BlockSpec((NCORE, _TC_ROWS, DH), lambda i: (0, i, 0)),
            pl.BlockSpec((_TC_ROWS, 1), lambda i: (i, 0)),
            pl.BlockSpec((D, D), lambda i: (0, 0)),
            pl.BlockSpec((1, D), lambda i: (0, 0)),
        ],
        out_specs=pl.BlockSpec((_TC_ROWS, D), lambda i: (i, 0)),
        out_shape=jax.ShapeDtypeStruct((N, D), jnp.float32),
    )(S, dinv.reshape(NPAD, 1)[:N], W, b.reshape(1, D))

    return out


# speedup vs baseline: 1.4537x; 1.0556x over previous
import functools

import jax
import jax.numpy as jnp
from jax import lax
from jax.experimental import pallas as pl
from jax.experimental.pallas import tpu as pltpu
from jax.experimental.pallas import tpu_sc as plsc

N = 10000
D = 128
DH = 64
E = 320000
NPAD = 10240
NCORE = 2
NSUB = 16
RPT = NPAD // NSUB
CHUNK = 128
ECH = 125
NCH = 160
NBUF = 5
QCH = 40
DCH = RPT // 64


def _sc_kernel_fn():
    mesh = plsc.VectorSubcoreMesh(core_axis_name="c", subcore_axis_name="s")

    @functools.partial(
        pl.kernel,
        mesh=mesh,
        compiler_params=pltpu.CompilerParams(
            use_tc_tiling_on_sc=False, needs_layout_passes=False),
        out_type=(
            jax.ShapeDtypeStruct((NCORE, NPAD, DH), jnp.float32),
            jax.ShapeDtypeStruct((NCORE, NPAD, DH), jnp.float32),
            jax.ShapeDtypeStruct((NPAD,), jnp.float32),
        ),
        scratch_types=[
            pltpu.VMEM((QCH, ECH), jnp.int32),
            pltpu.VMEM((QCH, ECH), jnp.int32),
            pltpu.VMEM((QCH, ECH), jnp.int32),
            pltpu.VMEM((QCH, ECH), jnp.int32),
            pltpu.VMEM((CHUNK, DH), jnp.float32),
            pltpu.VMEM((CHUNK, DH), jnp.float32),
            pltpu.VMEM((CHUNK, DH), jnp.float32),
            pltpu.VMEM((CHUNK, DH), jnp.float32),
            pltpu.VMEM((CHUNK, DH), jnp.float32),
            pltpu.VMEM((64, D), jnp.float32),
            pltpu.VMEM((64, D), jnp.float32),
            pltpu.VMEM((CHUNK,), jnp.float32),
            pltpu.VMEM((RPT,), jnp.float32),
            pltpu.VMEM_SHARED((NPAD,), jnp.float32),
            pltpu.VMEM_SHARED((NPAD, DH), jnp.float32),
            pltpu.SemaphoreType.DMA,
            pltpu.SemaphoreType.DMA,
            pltpu.SemaphoreType.DMA,
            pltpu.SemaphoreType.DMA,
            pltpu.SemaphoreType.DMA,
            pltpu.SemaphoreType.DMA,
            pltpu.SemaphoreType.DMA,
            pltpu.SemaphoreType.DMA,
            pltpu.SemaphoreType.DMA,
            pltpu.SemaphoreType.DMA,
            pltpu.SemaphoreType.DMA,
        ],
    )
    def sc_kernel(x_hbm, edges_hbm,
                  s_out, xs_out, dinv_out,
                  sq0, sq1, dq0, dq1, rb0, rb1, rb2, rb3, rb4,
                  xb0, xb1, ones_v, dloc_v, deg_sh, acc_sh,
                  dsem, gs0, gs1, gs2, gs3, gs4, ss0, ss1, ss2, ss3, ss4):
        c = lax.axis_index("c")
        s = lax.axis_index("s")
        row0 = s * RPT
        rbufs = [rb0, rb1, rb2, rb3, rb4]
        gsems = [gs0, gs1, gs2, gs3, gs4]
        ssems = [ss0, ss1, ss2, ss3, ss4]
        sqs = [sq0, sq1]
        dqs = [dq0, dq1]
        xbufs = [xb0, xb1]

        for j in range(8):
            ones_v[pl.ds(j * 16, 16)] = jnp.ones((16,), jnp.float32)

        def _zero_dloc(k, _):
            dloc_v[pl.ds(k * 16, 16)] = jnp.zeros((16,), jnp.float32)
            return 0
        lax.fori_loop(0, RPT // 16, _zero_dloc, 0)
        pltpu.sync_copy(dloc_v, deg_sh.at[pl.ds(row0, RPT)])
        plsc.subcore_barrier()

        _scB = jax.named_scope("phaseB_deg"); _scB.__enter__()
        for q in range(NCH // QCH):
            dq = dqs[q % 2]
            pltpu.sync_copy(
                edges_hbm.at[1, pl.ds(s * NCH + q * QCH, QCH)], dq)

            def _deg(i, _, dq=dq):
                pltpu.async_copy(
                    ones_v.at[pl.ds(0, ECH)], deg_sh.at[dq.at[i]],
                    dsem, add=True)

                @pl.when(i >= 8)
                def _():
                    pltpu.make_async_copy(
                        ones_v.at[pl.ds(0, ECH)], deg_sh.at[dq.at[i - 8]],
                        dsem).wait()
                return 0
            lax.fori_loop(0, QCH, _deg, 0)
            for i in range(QCH - 8, QCH):
                pltpu.make_async_copy(
                    ones_v.at[pl.ds(0, ECH)], deg_sh.at[dq.at[i]],
                    dsem).wait()
        plsc.subcore_barrier()
        _scB.__exit__(None, None, None)

        pltpu.sync_copy(deg_sh.at[pl.ds(row0, RPT)], dloc_v)

        def _dinv(k, _):
            dv = dloc_v[pl.ds(k * 16, 16)] + 1.0
            bits = lax.bitcast_convert_type(dv, jnp.int32)
            y = lax.bitcast_convert_type(
                jnp.int32(0x5F3759DF) - (bits >> 1), jnp.float32)
            half = dv * 0.5
            y = y * (1.5 - half * y * y)
            y = y * (1.5 - half * y * y)
            y = y * (1.5 - half * y * y)
            dloc_v[pl.ds(k * 16, 16)] = y
            return 0
        lax.fori_loop(0, RPT // 16, _dinv, 0)
        pltpu.sync_copy(dloc_v, dinv_out.at[pl.ds(row0, RPT)])

        _scD = jax.named_scope("phaseD_scale"); _scD.__enter__()
        in_sems = [gs0, gs1]
        o0_sems = [gs2, gs3]
        o1_sems = [ss0, ss1]
        acc_sems = [ss2, ss3]

        def _d_bufs(ch):
            p = ch % 2
            return rbufs[0].at[pl.ds(64 * p, 64)], rbufs[1].at[pl.ds(64 * p, 64)]

        def _d_in_start(ch):
            pltpu.async_copy(
                x_hbm.at[pl.ds(row0 + ch * 64, 64)],
                xbufs[ch % 2], in_sems[ch % 2])

        def _d_in_wait(ch):
            pltpu.make_async_copy(
                x_hbm.at[pl.ds(row0 + ch * 64, 64)],
                xbufs[ch % 2], in_sems[ch % 2]).wait()

        def _d_out_start(ch):
            r0 = row0 + ch * 64
            p = ch % 2
            oh0, oh1 = _d_bufs(ch)
            pltpu.async_copy(oh0, xs_out.at[0, pl.ds(r0, 64)], o0_sems[p])
            pltpu.async_copy(oh1, xs_out.at[1, pl.ds(r0, 64)], o1_sems[p])

            @pl.when(c == 0)
            def _():
                pltpu.async_copy(
                    oh0, acc_sh.at[pl.ds(r0, 64)], acc_sems[p])

            @pl.when(c == 1)
            def _():
                pltpu.async_copy(
                    oh1, acc_sh.at[pl.ds(r0, 64)], acc_sems[p])

        def _d_out_wait(ch):
            r0 = row0 + ch * 64
            p = ch % 2
            oh0, oh1 = _d_bufs(ch)
            pltpu.make_async_copy(
                oh0, xs_out.at[0, pl.ds(r0, 64)], o0_sems[p]).wait()
            pltpu.make_async_copy(
                oh1, xs_out.at[1, pl.ds(r0, 64)], o1_sems[p]).wait()

            @pl.when(c == 0)
            def _():
                pltpu.make_async_copy(
                    oh0, acc_sh.at[pl.ds(r0, 64)], acc_sems[p]).wait()

            @pl.when(c == 1)
            def _():
                pltpu.make_async_copy(
                    oh1, acc_sh.at[pl.ds(r0, 64)], acc_sems[p]).wait()

        _d_in_start(0)
        for ch in range(DCH):
            if ch + 1 < DCH:
                _d_in_start(ch + 1)
            _d_in_wait(ch)
            if ch >= 2:
                _d_out_wait(ch - 2)
            p = ch % 2
            xin = xbufs[p]
            ob0, ob1 = rbufs[0], rbufs[1]

            def _row(rr, _, ch=ch, xin=xin, p=p, ob0=ob0, ob1=ob1):
                idxv = jnp.zeros((16,), jnp.int32) + (ch * 64 + rr)
                dvb = plsc.load_gather(dloc_v, [idxv])
                for j in range(4):
                    sl = pl.ds(j * 16, 16)
                    ob0[64 * p + rr, sl] = xin[rr, sl] * dvb
                for j in range(4):
                    sl = pl.ds(j * 16, 16)
                    ob1[64 * p + rr, sl] = xin[rr, pl.ds(64 + j * 16, 16)] * dvb
                return 0
            lax.fori_loop(0, 64, _row, 0)
            _d_out_start(ch)
        _d_out_wait(DCH - 2)
        _d_out_wait(DCH - 1)
        plsc.subcore_barrier()
        _scD.__exit__(None, None, None)
        _scE = jax.named_scope("phaseE_main"); _scE.__enter__()

        def _gstart(k, sq, i):
            pltpu.async_copy(
                xs_out.at[c].at[sq.at[i]],
                rbufs[k].at[pl.ds(0, ECH)], gsems[k])

        def _gwait(k, sq, i):
            pltpu.make_async_copy(
                xs_out.at[c].at[sq.at[i]],
                rbufs[k].at[pl.ds(0, ECH)], gsems[k]).wait()

        def _sstart(k, dq, i):
            pltpu.async_copy(
                rbufs[k].at[pl.ds(0, ECH)], acc_sh.at[dq.at[i]],
                ssems[k], add=True)

        def _swait(k, dq, i):
            pltpu.make_async_copy(
                rbufs[k].at[pl.ds(0, ECH)], acc_sh.at[dq.at[i]],
                ssems[k]).wait()

        for q in range(NCH // QCH):
            sq = sqs[q % 2]
            dq = dqs[q % 2]
            pltpu.sync_copy(
                edges_hbm.at[0, pl.ds(s * NCH + q * QCH, QCH)], sq)
            pltpu.sync_copy(
                edges_hbm.at[1, pl.ds(s * NCH + q * QCH, QCH)], dq)
            for k in range(NBUF):
                _gstart(k, sq, k)

            def _quad(p, _, sq=sq, dq=dq):
                i0 = NBUF * p
                for k in range(NBUF):
                    i = i0 + k
                    _gwait(k, sq, i)
                    _sstart(k, dq, i)
                    km1 = (k - 1) % NBUF
                    if km1 == NBUF - 1:
                        @pl.when(p > 0)
                        def _():
                            _swait(km1, dq, i - 1)
                            _gstart(km1, sq, i + NBUF - 1)
                    else:
                        @pl.when(i + NBUF - 1 < QCH)
                        def _():
                            _swait(km1, dq, i - 1)
                            _gstart(km1, sq, i + NBUF - 1)
                return 0
            lax.fori_loop(0, QCH // NBUF, _quad, 0)
            for k in range(NBUF - 1):
                _swait(k, dq, QCH - NBUF + k)
            _swait(NBUF - 1, dq, QCH - 1)
        plsc.subcore_barrier()
        _scE.__exit__(None, None, None)

        def _out(k, _):
            r0 = row0 + k * CHUNK
            pltpu.sync_copy(acc_sh.at[pl.ds(r0, CHUNK)], rb0)
            pltpu.sync_copy(rb0, s_out.at[c, pl.ds(r0, CHUNK)])
            return 0
        lax.fori_loop(0, RPT // CHUNK, _out, 0)

    return sc_kernel


_SC_KERNEL = _sc_kernel_fn()

_TC_ROWS = 2000


def _tc_body(s_ref, dinv_ref, w_ref, b_ref, o_ref):
    u = jnp.concatenate([s_ref[0], s_ref[1]], axis=1)
    acc = jnp.dot(u, w_ref[...], preferred_element_type=jnp.float32)
    o_ref[...] = acc * dinv_ref[...] + b_ref[...]


def kernel(x, edge_index, W, b):
    edges = edge_index.astype(jnp.int32).reshape(2, NSUB * NCH, ECH)
    x_pad = jnp.concatenate(
        [x, jnp.zeros((NPAD - N, D), jnp.float32)], axis=0)

    S, _, dinv = _SC_KERNEL(x_pad, edges)

    out = pl.pallas_call(
        _tc_body,
        grid=(N // _TC_ROWS,),
        in_specs=[
            pl.BlockSpec((NCORE, _TC_ROWS, DH), lambda i: (0, i, 0)),
            pl.BlockSpec((_TC_ROWS, 1), lambda i: (i, 0)),
            pl.BlockSpec((D, D), lambda i: (0, 0)),
            pl.BlockSpec((1, D), lambda i: (0, 0)),
        ],
        out_specs=pl.BlockSpec((_TC_ROWS, D), lambda i: (i, 0)),
        out_shape=jax.ShapeDtypeStruct((N, D), jnp.float32),
    )(S, dinv.reshape(NPAD, 1)[:N], W, b.reshape(1, D))

    return out


# speedup vs baseline: 1.4966x; 1.0295x over previous
import functools

import jax
import jax.numpy as jnp
from jax import lax
from jax.experimental import pallas as pl
from jax.experimental.pallas import tpu as pltpu
from jax.experimental.pallas import tpu_sc as plsc

N = 10000
D = 128
DH = 64
E = 320000
NPAD = 10240
NCORE = 2
NSUB = 16
RPT = NPAD // NSUB
CHUNK = 128
NROWS = E // CHUNK
BCH = 156
NXTRA = NROWS - NSUB * BCH
QCH = 52
NBUF = 4
DCH = RPT // 64


def _sc_kernel_fn():
    mesh = plsc.VectorSubcoreMesh(core_axis_name="c", subcore_axis_name="s")

    @functools.partial(
        pl.kernel,
        mesh=mesh,
        compiler_params=pltpu.CompilerParams(
            use_tc_tiling_on_sc=False, needs_layout_passes=False),
        out_type=(
            jax.ShapeDtypeStruct((NCORE, NPAD, DH), jnp.float32),
            jax.ShapeDtypeStruct((NCORE, NPAD, DH), jnp.float32),
        ),
        scratch_types=[
            pltpu.VMEM((QCH, CHUNK), jnp.int32),
            pltpu.VMEM((QCH, CHUNK), jnp.int32),
            pltpu.VMEM((QCH, CHUNK), jnp.int32),
            pltpu.VMEM((QCH, CHUNK), jnp.int32),
            pltpu.VMEM((CHUNK, DH), jnp.float32),
            pltpu.VMEM((CHUNK, DH), jnp.float32),
            pltpu.VMEM((CHUNK, DH), jnp.float32),
            pltpu.VMEM((CHUNK, DH), jnp.float32),
            pltpu.VMEM((64, D), jnp.float32),
            pltpu.VMEM((64, D), jnp.float32),
            pltpu.VMEM((CHUNK,), jnp.float32),
            pltpu.VMEM((RPT,), jnp.float32),
            pltpu.VMEM_SHARED((NPAD,), jnp.float32),
            pltpu.VMEM_SHARED((NPAD, DH), jnp.float32),
            pltpu.SemaphoreType.DMA,
            pltpu.SemaphoreType.DMA,
            pltpu.SemaphoreType.DMA,
            pltpu.SemaphoreType.DMA,
            pltpu.SemaphoreType.DMA,
            pltpu.SemaphoreType.DMA,
            pltpu.SemaphoreType.DMA,
            pltpu.SemaphoreType.DMA,
            pltpu.SemaphoreType.DMA,
        ],
    )
    def sc_kernel(x_hbm, edges_hbm,
                  s_out, xs_out,
                  sq0, sq1, dq0, dq1, rb0, rb1, rb2, rb3,
                  xb0, xb1, ones_v, dloc_v, deg_sh, acc_sh,
                  dsem, gs0, gs1, gs2, gs3, ss0, ss1, ss2, ss3):
        c = lax.axis_index("c")
        s = lax.axis_index("s")
        row0 = s * RPT
        rbufs = [rb0, rb1, rb2, rb3]
        gsems = [gs0, gs1, gs2, gs3]
        ssems = [ss0, ss1, ss2, ss3]
        sqs = [sq0, sq1]
        dqs = [dq0, dq1]
        xbufs = [xb0, xb1]
        cbase = s * BCH + jnp.maximum(s - (NSUB - NXTRA), 0)
        has_extra = s >= (NSUB - NXTRA)

        for j in range(8):
            ones_v[pl.ds(j * 16, 16)] = jnp.ones((16,), jnp.float32)

        def _zero_dloc(k, _):
            dloc_v[pl.ds(k * 16, 16)] = jnp.zeros((16,), jnp.float32)
            return 0
        lax.fori_loop(0, RPT // 16, _zero_dloc, 0)
        pltpu.sync_copy(dloc_v, deg_sh.at[pl.ds(row0, RPT)])
        plsc.subcore_barrier()

        _scB = jax.named_scope("phaseB_deg"); _scB.__enter__()
        for q in range(BCH // QCH):
            dq = dqs[q % 2]
            pltpu.sync_copy(
                edges_hbm.at[pl.ds(NROWS + cbase + q * QCH, QCH)], dq)

            def _deg(i, _, dq=dq):
                pltpu.async_copy(ones_v, deg_sh.at[dq.at[i]], dsem, add=True)

                @pl.when(i >= 8)
                def _():
                    pltpu.make_async_copy(
                        ones_v, deg_sh.at[dq.at[i - 8]], dsem).wait()
                return 0
            lax.fori_loop(0, QCH, _deg, 0)
            for i in range(QCH - 8, QCH):
                pltpu.make_async_copy(
                    ones_v, deg_sh.at[dq.at[i]], dsem).wait()

        @pl.when(has_extra)
        def _():
            pltpu.sync_copy(
                edges_hbm.at[pl.ds(NROWS + cbase + BCH, 1)],
                dqs[0].at[pl.ds(0, 1)])
            pltpu.sync_copy(ones_v, deg_sh.at[dqs[0].at[0]], add=True)
        plsc.subcore_barrier()
        _scB.__exit__(None, None, None)

        pltpu.sync_copy(deg_sh.at[pl.ds(row0, RPT)], dloc_v)

        def _dinv(k, _):
            dv = dloc_v[pl.ds(k * 16, 16)] + 1.0
            bits = lax.bitcast_convert_type(dv, jnp.int32)
            y = lax.bitcast_convert_type(
                jnp.int32(0x5F3759DF) - (bits >> 1), jnp.float32)
            half = dv * 0.5
            y = y * (1.5 - half * y * y)
            y = y * (1.5 - half * y * y)
            y = y * (1.5 - half * y * y)
            dloc_v[pl.ds(k * 16, 16)] = y
            return 0
        lax.fori_loop(0, RPT // 16, _dinv, 0)

        _scD = jax.named_scope("phaseD_scale"); _scD.__enter__()
        in_sems = [gs0, gs1]
        o0_sems = [gs2, gs3]
        o1_sems = [ss0, ss1]
        acc_sems = [ss2, ss3]

        def _d_bufs(ch):
            p = ch % 2
            return (rbufs[0].at[pl.ds(64 * p, 64)],
                    rbufs[1].at[pl.ds(64 * p, 64)])

        def _d_in_start(ch):
            pltpu.async_copy(
                x_hbm.at[pl.ds(row0 + ch * 64, 64)],
                xbufs[ch % 2], in_sems[ch % 2])

        def _d_in_wait(ch):
            pltpu.make_async_copy(
                x_hbm.at[pl.ds(row0 + ch * 64, 64)],
                xbufs[ch % 2], in_sems[ch % 2]).wait()

        def _d_out_start(ch):
            r0 = row0 + ch * 64
            p = ch % 2
            oh0, oh1 = _d_bufs(ch)
            pltpu.async_copy(oh0, xs_out.at[0, pl.ds(r0, 64)], o0_sems[p])
            pltpu.async_copy(oh1, xs_out.at[1, pl.ds(r0, 64)], o1_sems[p])

            @pl.when(c == 0)
            def _():
                pltpu.async_copy(
                    oh0, acc_sh.at[pl.ds(r0, 64)], acc_sems[p])

            @pl.when(c == 1)
            def _():
                pltpu.async_copy(
                    oh1, acc_sh.at[pl.ds(r0, 64)], acc_sems[p])

        def _d_out_wait(ch):
            r0 = row0 + ch * 64
            p = ch % 2
            oh0, oh1 = _d_bufs(ch)
            pltpu.make_async_copy(
                oh0, xs_out.at[0, pl.ds(r0, 64)], o0_sems[p]).wait()
            pltpu.make_async_copy(
                oh1, xs_out.at[1, pl.ds(r0, 64)], o1_sems[p]).wait()

            @pl.when(c == 0)
            def _():
                pltpu.make_async_copy(
                    oh0, acc_sh.at[pl.ds(r0, 64)], acc_sems[p]).wait()

            @pl.when(c == 1)
            def _():
                pltpu.make_async_copy(
                    oh1, acc_sh.at[pl.ds(r0, 64)], acc_sems[p]).wait()

        _d_in_start(0)
        for ch in range(DCH):
            if ch + 1 < DCH:
                _d_in_start(ch + 1)
            _d_in_wait(ch)
            if ch >= 2:
                _d_out_wait(ch - 2)
            p = ch % 2
            xin = xbufs[p]
            ob0, ob1 = rbufs[0], rbufs[1]

            def _row(rr, _, ch=ch, xin=xin, p=p, ob0=ob0, ob1=ob1):
                idxv = jnp.zeros((16,), jnp.int32) + (ch * 64 + rr)
                dvb = plsc.load_gather(dloc_v, [idxv])
                for j in range(4):
                    sl = pl.ds(j * 16, 16)
                    ob0[64 * p + rr, sl] = xin[rr, sl] * dvb
                for j in range(4):
                    sl = pl.ds(j * 16, 16)
                    ob1[64 * p + rr, sl] = xin[rr, pl.ds(64 + j * 16, 16)] * dvb
                return 0
            lax.fori_loop(0, 64, _row, 0)
            _d_out_start(ch)
        _d_out_wait(DCH - 2)
        _d_out_wait(DCH - 1)
        plsc.subcore_barrier()
        _scD.__exit__(None, None, None)
        _scE = jax.named_scope("phaseE_main"); _scE.__enter__()

        def _gstart(k, sq, i):
            pltpu.async_copy(
                xs_out.at[c].at[sq.at[i]], rbufs[k], gsems[k])

        def _gwait(k, sq, i):
            pltpu.make_async_copy(
                xs_out.at[c].at[sq.at[i]], rbufs[k], gsems[k]).wait()

        def _sstart(k, dq, i):
            pltpu.async_copy(
                rbufs[k], acc_sh.at[dq.at[i]], ssems[k], add=True)

        def _swait(k, dq, i):
            pltpu.make_async_copy(
                rbufs[k], acc_sh.at[dq.at[i]], ssems[k]).wait()

        for q in range(BCH // QCH):
            sq = sqs[q % 2]
            dq = dqs[q % 2]
            pltpu.sync_copy(
                edges_hbm.at[pl.ds(cbase + q * QCH, QCH)], sq)
            pltpu.sync_copy(
                edges_hbm.at[pl.ds(NROWS + cbase + q * QCH, QCH)], dq)
            for k in range(NBUF):
                _gstart(k, sq, k)

            def _quad(p, _, sq=sq, dq=dq):
                i0 = NBUF * p
                for k in range(NBUF):
                    i = i0 + k
                    _gwait(k, sq, i)
                    _sstart(k, dq, i)
                    km1 = (k - 1) % NBUF
                    if km1 == NBUF - 1:
                        @pl.when(p > 0)
                        def _():
                            _swait(km1, dq, i - 1)
                            _gstart(km1, sq, i + NBUF - 1)
                    else:
                        @pl.when(i + NBUF - 1 < QCH)
                        def _():
                            _swait(km1, dq, i - 1)
                            _gstart(km1, sq, i + NBUF - 1)
                return 0
            lax.fori_loop(0, QCH // NBUF, _quad, 0)
            for k in range(NBUF - 1):
                _swait(k, dq, QCH - NBUF + k)
            _swait(NBUF - 1, dq, QCH - 1)

        @pl.when(has_extra)
        def _():
            pltpu.sync_copy(
                edges_hbm.at[pl.ds(cbase + BCH, 1)], sqs[0].at[pl.ds(0, 1)])
            pltpu.sync_copy(
                edges_hbm.at[pl.ds(NROWS + cbase + BCH, 1)],
                dqs[0].at[pl.ds(0, 1)])
            pltpu.async_copy(
                xs_out.at[c].at[sqs[0].at[0]], rbufs[0], gsems[0])
            pltpu.make_async_copy(
                xs_out.at[c].at[sqs[0].at[0]], rbufs[0], gsems[0]).wait()
            pltpu.sync_copy(rbufs[0], acc_sh.at[dqs[0].at[0]], add=True)
        plsc.subcore_barrier()
        _scE.__exit__(None, None, None)

        def _f_in(k):
            pltpu.sync_copy(
                acc_sh.at[pl.ds(row0 + k * CHUNK, CHUNK)], rbufs[k % 2])

        def _f_out_start(k):
            pltpu.async_copy(
                rbufs[k % 2],
                s_out.at[c, pl.ds(row0 + k * CHUNK, CHUNK)], gsems[k % 2])

        def _f_out_wait(k):
            pltpu.make_async_copy(
                rbufs[k % 2],
                s_out.at[c, pl.ds(row0 + k * CHUNK, CHUNK)],
                gsems[k % 2]).wait()

        for k in range(RPT // CHUNK):
            if k >= 2:
                _f_out_wait(k - 2)
            _f_in(k)
            rbk = rbufs[k % 2]

            def _frow(rr, _, k=k, rbk=rbk):
                idxv = jnp.zeros((16,), jnp.int32) + (k * CHUNK + rr)
                dvb = plsc.load_gather(dloc_v, [idxv])
                for j in range(4):
                    sl = pl.ds(j * 16, 16)
                    rbk[rr, sl] = rbk[rr, sl] * dvb
                return 0
            lax.fori_loop(0, CHUNK, _frow, 0)
            _f_out_start(k)
        _f_out_wait(RPT // CHUNK - 2)
        _f_out_wait(RPT // CHUNK - 1)

    return sc_kernel


_SC_KERNEL = _sc_kernel_fn()

_TC_ROWS = 2000


def _tc_body(s_ref, w_ref, b_ref, o_ref):
    u = jnp.concatenate([s_ref[0], s_ref[1]], axis=1)
    acc = jnp.dot(u, w_ref[...], preferred_element_type=jnp.float32)
    o_ref[...] = acc + b_ref[...]


def kernel(x, edge_index, W, b):
    edges = edge_index.astype(jnp.int32).reshape(2 * NROWS, CHUNK)
    x_pad = jnp.concatenate(
        [x, jnp.zeros((NPAD - N, D), jnp.float32)], axis=0)

    S, _ = _SC_KERNEL(x_pad, edges)

    out = pl.pallas_call(
        _tc_body,
        grid=(N // _TC_ROWS,),
        in_specs=[
            pl.BlockSpec((NCORE, _TC_ROWS, DH), lambda i: (0, i, 0)),
            pl.BlockSpec((D, D), lambda i: (0, 0)),
            pl.BlockSpec((1, D), lambda i: (0, 0)),
        ],
        out_specs=pl.BlockSpec((_TC_ROWS, D), lambda i: (i, 0)),
        out_shape=jax.ShapeDtypeStruct((N, D), jnp.float32),
    )(S, W, b.reshape(1, D))

    return out


# speedup vs baseline: 1.4972x; 1.0004x over previous
import functools

import jax
import jax.numpy as jnp
from jax import lax
from jax.experimental import pallas as pl
from jax.experimental.pallas import tpu as pltpu
from jax.experimental.pallas import tpu_sc as plsc

N = 10000
D = 128
DH = 64
E = 320000
NPAD = 10240
NCORE = 2
NSUB = 16
RPT = NPAD // NSUB
CHUNK = 128
NROWS = E // CHUNK
BCH = 156
NXTRA = NROWS - NSUB * BCH
QCH = 52
NBUF = 4
DCH = RPT // 64


def _sc_kernel_fn():
    mesh = plsc.VectorSubcoreMesh(core_axis_name="c", subcore_axis_name="s")

    @functools.partial(
        pl.kernel,
        mesh=mesh,
        compiler_params=pltpu.CompilerParams(
            use_tc_tiling_on_sc=False, needs_layout_passes=False),
        out_type=(
            jax.ShapeDtypeStruct((NCORE, NPAD, DH), jnp.float32),
            jax.ShapeDtypeStruct((NCORE, NPAD, DH), jnp.float32),
        ),
        scratch_types=[
            pltpu.VMEM((QCH, CHUNK), jnp.int32),
            pltpu.VMEM((QCH, CHUNK), jnp.int32),
            pltpu.VMEM((QCH, CHUNK), jnp.int32),
            pltpu.VMEM((QCH, CHUNK), jnp.int32),
            pltpu.VMEM((CHUNK, DH), jnp.float32),
            pltpu.VMEM((CHUNK, DH), jnp.float32),
            pltpu.VMEM((CHUNK, DH), jnp.float32),
            pltpu.VMEM((CHUNK, DH), jnp.float32),
            pltpu.VMEM((64, D), jnp.float32),
            pltpu.VMEM((64, D), jnp.float32),
            pltpu.VMEM((CHUNK,), jnp.float32),
            pltpu.VMEM((RPT,), jnp.float32),
            pltpu.VMEM_SHARED((NPAD,), jnp.float32),
            pltpu.VMEM_SHARED((NPAD, DH), jnp.float32),
            pltpu.SemaphoreType.DMA,
            pltpu.SemaphoreType.DMA,
            pltpu.SemaphoreType.DMA,
            pltpu.SemaphoreType.DMA,
            pltpu.SemaphoreType.DMA,
            pltpu.SemaphoreType.DMA,
            pltpu.SemaphoreType.DMA,
            pltpu.SemaphoreType.DMA,
            pltpu.SemaphoreType.DMA,
        ],
    )
    def sc_kernel(x_hbm, edges_hbm,
                  s_out, xs_out,
                  sq0, sq1, dq0, dq1, rb0, rb1, rb2, rb3,
                  xb0, xb1, ones_v, dloc_v, deg_sh, acc_sh,
                  dsem, gs0, gs1, gs2, gs3, ss0, ss1, ss2, ss3):
        c = lax.axis_index("c")
        s = lax.axis_index("s")
        row0 = s * RPT
        rbufs = [rb0, rb1, rb2, rb3]
        gsems = [gs0, gs1, gs2, gs3]
        ssems = [ss0, ss1, ss2, ss3]
        sqs = [sq0, sq1]
        dqs = [dq0, dq1]
        xbufs = [xb0, xb1]
        cbase = s * BCH + jnp.maximum(s - (NSUB - NXTRA), 0)
        has_extra = s >= (NSUB - NXTRA)

        for j in range(8):
            ones_v[pl.ds(j * 16, 16)] = jnp.ones((16,), jnp.float32)

        def _zero_dloc(k, _):
            dloc_v[pl.ds(k * 16, 16)] = jnp.zeros((16,), jnp.float32)
            return 0
        lax.fori_loop(0, RPT // 16, _zero_dloc, 0)
        pltpu.sync_copy(dloc_v, deg_sh.at[pl.ds(row0, RPT)])
        plsc.subcore_barrier()

        _scB = jax.named_scope("phaseB_deg"); _scB.__enter__()
        for q in range(BCH // QCH):
            dq = dqs[q % 2]
            pltpu.sync_copy(
                edges_hbm.at[pl.ds(NROWS + cbase + q * QCH, QCH)], dq)

            def _deg(i, _, dq=dq):
                pltpu.async_copy(ones_v, deg_sh.at[dq.at[i]], dsem, add=True)

                @pl.when(i >= 8)
                def _():
                    pltpu.make_async_copy(
                        ones_v, deg_sh.at[dq.at[i - 8]], dsem).wait()
                return 0
            lax.fori_loop(0, QCH, _deg, 0)
            for i in range(QCH - 8, QCH):
                pltpu.make_async_copy(
                    ones_v, deg_sh.at[dq.at[i]], dsem).wait()

        @pl.when(has_extra)
        def _():
            pltpu.sync_copy(
                edges_hbm.at[pl.ds(NROWS + cbase + BCH, 1)],
                dqs[0].at[pl.ds(0, 1)])
            pltpu.sync_copy(ones_v, deg_sh.at[dqs[0].at[0]], add=True)
        plsc.subcore_barrier()
        _scB.__exit__(None, None, None)

        pltpu.sync_copy(deg_sh.at[pl.ds(row0, RPT)], dloc_v)

        def _dinv(k, _):
            dv = dloc_v[pl.ds(k * 16, 16)] + 1.0
            bits = lax.bitcast_convert_type(dv, jnp.int32)
            y = lax.bitcast_convert_type(
                jnp.int32(0x5F3759DF) - (bits >> 1), jnp.float32)
            half = dv * 0.5
            y = y * (1.5 - half * y * y)
            y = y * (1.5 - half * y * y)
            y = y * (1.5 - half * y * y)
            dloc_v[pl.ds(k * 16, 16)] = y
            return 0
        lax.fori_loop(0, RPT // 16, _dinv, 0)

        _scD = jax.named_scope("phaseD_scale"); _scD.__enter__()
        in_sems = [gs0, gs1]
        o0_sems = [gs2, gs3]
        o1_sems = [ss0, ss1]
        acc_sems = [ss2, ss3]

        def _d_bufs(ch):
            p = ch % 2
            return (rbufs[0].at[pl.ds(64 * p, 64)],
                    rbufs[1].at[pl.ds(64 * p, 64)])

        def _d_in_start(ch):
            pltpu.async_copy(
                x_hbm.at[pl.ds(row0 + ch * 64, 64)],
                xbufs[ch % 2], in_sems[ch % 2])

        def _d_in_wait(ch):
            pltpu.make_async_copy(
                x_hbm.at[pl.ds(row0 + ch * 64, 64)],
                xbufs[ch % 2], in_sems[ch % 2]).wait()

        def _d_out_start(ch):
            r0 = row0 + ch * 64
            p = ch % 2
            oh0, oh1 = _d_bufs(ch)
            pltpu.async_copy(oh0, xs_out.at[0, pl.ds(r0, 64)], o0_sems[p])
            pltpu.async_copy(oh1, xs_out.at[1, pl.ds(r0, 64)], o1_sems[p])

            @pl.when(c == 0)
            def _():
                pltpu.async_copy(
                    oh0, acc_sh.at[pl.ds(r0, 64)], acc_sems[p])

            @pl.when(c == 1)
            def _():
                pltpu.async_copy(
                    oh1, acc_sh.at[pl.ds(r0, 64)], acc_sems[p])

        def _d_out_wait(ch):
            r0 = row0 + ch * 64
            p = ch % 2
            oh0, oh1 = _d_bufs(ch)
            pltpu.make_async_copy(
                oh0, xs_out.at[0, pl.ds(r0, 64)], o0_sems[p]).wait()
            pltpu.make_async_copy(
                oh1, xs_out.at[1, pl.ds(r0, 64)], o1_sems[p]).wait()

            @pl.when(c == 0)
            def _():
                pltpu.make_async_copy(
                    oh0, acc_sh.at[pl.ds(r0, 64)], acc_sems[p]).wait()

            @pl.when(c == 1)
            def _():
                pltpu.make_async_copy(
                    oh1, acc_sh.at[pl.ds(r0, 64)], acc_sems[p]).wait()

        _d_in_start(0)
        for ch in range(DCH):
            if ch + 1 < DCH:
                _d_in_start(ch + 1)
            _d_in_wait(ch)
            if ch >= 2:
                _d_out_wait(ch - 2)
            p = ch % 2
            xin = xbufs[p]
            ob0, ob1 = rbufs[0], rbufs[1]

            def _row(g, _, ch=ch, xin=xin, p=p, ob0=ob0, ob1=ob1):
                dvbs = []
                for u in range(4):
                    idxv = jnp.zeros((16,), jnp.int32) + (ch * 64 + g * 4 + u)
                    dvbs.append(plsc.load_gather(dloc_v, [idxv]))
                for u in range(4):
                    rr = g * 4 + u
                    for j in range(4):
                        sl = pl.ds(j * 16, 16)
                        ob0[64 * p + rr, sl] = xin[rr, sl] * dvbs[u]
                    for j in range(4):
                        sl = pl.ds(j * 16, 16)
                        ob1[64 * p + rr, sl] = (
                            xin[rr, pl.ds(64 + j * 16, 16)] * dvbs[u])
                return 0
            lax.fori_loop(0, 16, _row, 0)
            _d_out_start(ch)
        _d_out_wait(DCH - 2)
        _d_out_wait(DCH - 1)
        plsc.subcore_barrier()
        _scD.__exit__(None, None, None)
        _scE = jax.named_scope("phaseE_main"); _scE.__enter__()

        def _gstart(k, sq, i):
            pltpu.async_copy(
                xs_out.at[c].at[sq.at[i]], rbufs[k], gsems[k])

        def _gwait(k, sq, i):
            pltpu.make_async_copy(
                xs_out.at[c].at[sq.at[i]], rbufs[k], gsems[k]).wait()

        def _sstart(k, dq, i):
            pltpu.async_copy(
                rbufs[k], acc_sh.at[dq.at[i]], ssems[k], add=True)

        def _swait(k, dq, i):
            pltpu.make_async_copy(
                rbufs[k], acc_sh.at[dq.at[i]], ssems[k]).wait()

        for q in range(BCH // QCH):
            sq = sqs[q % 2]
            dq = dqs[q % 2]
            pltpu.sync_copy(
                edges_hbm.at[pl.ds(cbase + q * QCH, QCH)], sq)
            pltpu.sync_copy(
                edges_hbm.at[pl.ds(NROWS + cbase + q * QCH, QCH)], dq)
            for k in range(NBUF):
                _gstart(k, sq, k)

            def _quad(p, _, sq=sq, dq=dq):
                i0 = NBUF * p
                for k in range(NBUF):
                    i = i0 + k
                    _gwait(k, sq, i)
                    _sstart(k, dq, i)
                    km1 = (k - 1) % NBUF
                    if km1 == NBUF - 1:
                        @pl.when(p > 0)
                        def _():
                            _swait(km1, dq, i - 1)
                            _gstart(km1, sq, i + NBUF - 1)
                    else:
                        @pl.when(i + NBUF - 1 < QCH)
                        def _():
                            _swait(km1, dq, i - 1)
                            _gstart(km1, sq, i + NBUF - 1)
                return 0
            lax.fori_loop(0, QCH // NBUF, _quad, 0)
            for k in range(NBUF - 1):
                _swait(k, dq, QCH - NBUF + k)
            _swait(NBUF - 1, dq, QCH - 1)

        @pl.when(has_extra)
        def _():
            pltpu.sync_copy(
                edges_hbm.at[pl.ds(cbase + BCH, 1)], sqs[0].at[pl.ds(0, 1)])
            pltpu.sync_copy(
                edges_hbm.at[pl.ds(NROWS + cbase + BCH, 1)],
                dqs[0].at[pl.ds(0, 1)])
            pltpu.async_copy(
                xs_out.at[c].at[sqs[0].at[0]], rbufs[0], gsems[0])
            pltpu.make_async_copy(
                xs_out.at[c].at[sqs[0].at[0]], rbufs[0], gsems[0]).wait()
            pltpu.sync_copy(rbufs[0], acc_sh.at[dqs[0].at[0]], add=True)
        plsc.subcore_barrier()
        _scE.__exit__(None, None, None)

        def _f_in(k):
            pltpu.sync_copy(
                acc_sh.at[pl.ds(row0 + k * CHUNK, CHUNK)], rbufs[k % 2])

        def _f_out_start(k):
            pltpu.async_copy(
                rbufs[k % 2],
                s_out.at[c, pl.ds(row0 + k * CHUNK, CHUNK)], gsems[k % 2])

        def _f_out_wait(k):
            pltpu.make_async_copy(
                rbufs[k % 2],
                s_out.at[c, pl.ds(row0 + k * CHUNK, CHUNK)],
                gsems[k % 2]).wait()

        for k in range(RPT // CHUNK):
            if k >= 2:
                _f_out_wait(k - 2)
            _f_in(k)
            rbk = rbufs[k % 2]

            def _frow(g, _, k=k, rbk=rbk):
                dvbs = []
                for u in range(4):
                    idxv = jnp.zeros((16,), jnp.int32) + (k * CHUNK + g * 4 + u)
                    dvbs.append(plsc.load_gather(dloc_v, [idxv]))
                for u in range(4):
                    rr = g * 4 + u
                    for j in range(4):
                        sl = pl.ds(j * 16, 16)
                        rbk[rr, sl] = rbk[rr, sl] * dvbs[u]
                return 0
            lax.fori_loop(0, CHUNK // 4, _frow, 0)
            _f_out_start(k)
        _f_out_wait(RPT // CHUNK - 2)
        _f_out_wait(RPT // CHUNK - 1)

    return sc_kernel


_SC_KERNEL = _sc_kernel_fn()

_TC_ROWS = 2000


def _tc_body(s_ref, w_ref, b_ref, o_ref):
    u = jnp.concatenate([s_ref[0], s_ref[1]], axis=1)
    acc = jnp.dot(u, w_ref[...], preferred_element_type=jnp.float32)
    o_ref[...] = acc + b_ref[...]


def kernel(x, edge_index, W, b):
    edges = edge_index.astype(jnp.int32).reshape(2 * NROWS, CHUNK)
    x_pad = jnp.concatenate(
        [x, jnp.zeros((NPAD - N, D), jnp.float32)], axis=0)

    S, _ = _SC_KERNEL(x_pad, edges)

    out = pl.pallas_call(
        _tc_body,
        grid=(N // _TC_ROWS,),
        in_specs=[
            pl.BlockSpec((NCORE, _TC_ROWS, DH), lambda i: (0, i, 0)),
            pl.BlockSpec((D, D), lambda i: (0, 0)),
            pl.BlockSpec((1, D), lambda i: (0, 0)),
        ],
        out_specs=pl.BlockSpec((_TC_ROWS, D), lambda i: (i, 0)),
        out_shape=jax.ShapeDtypeStruct((N, D), jnp.float32),
    )(S, W, b.reshape(1, D))

    return out


# speedup vs baseline: 1.5072x; 1.0067x over previous
import functools

import jax
import jax.numpy as jnp
from jax import lax
from jax.experimental import pallas as pl
from jax.experimental.pallas import tpu as pltpu
from jax.experimental.pallas import tpu_sc as plsc

N = 10000
D = 128
DH = 64
E = 320000
NPAD = 10240
NCORE = 2
NSUB = 16
RPT = NPAD // NSUB
CHUNK = 128
NROWS = E // CHUNK
BCH = 156
NXTRA = NROWS - NSUB * BCH
QCH = 52
NBUF = 4
DCH = RPT // 64


def _sc_kernel_fn():
    mesh = plsc.VectorSubcoreMesh(core_axis_name="c", subcore_axis_name="s")

    @functools.partial(
        pl.kernel,
        mesh=mesh,
        compiler_params=pltpu.CompilerParams(
            use_tc_tiling_on_sc=False, needs_layout_passes=False),
        out_type=(
            jax.ShapeDtypeStruct((NCORE, NPAD, DH), jnp.float32),
            jax.ShapeDtypeStruct((NCORE, NPAD, DH), jnp.float32),
        ),
        scratch_types=[
            pltpu.VMEM((QCH, CHUNK), jnp.int32),
            pltpu.VMEM((QCH, CHUNK), jnp.int32),
            pltpu.VMEM((QCH, CHUNK), jnp.int32),
            pltpu.VMEM((QCH, CHUNK), jnp.int32),
            pltpu.VMEM((CHUNK, DH), jnp.float32),
            pltpu.VMEM((CHUNK, DH), jnp.float32),
            pltpu.VMEM((CHUNK, DH), jnp.float32),
            pltpu.VMEM((CHUNK, DH), jnp.float32),
            pltpu.VMEM((64, D), jnp.float32),
            pltpu.VMEM((64, D), jnp.float32),
            pltpu.VMEM((CHUNK,), jnp.float32),
            pltpu.VMEM((RPT,), jnp.float32),
            pltpu.VMEM_SHARED((NPAD,), jnp.float32),
            pltpu.VMEM_SHARED((NPAD, DH), jnp.float32),
            pltpu.SemaphoreType.DMA,
            pltpu.SemaphoreType.DMA,
            pltpu.SemaphoreType.DMA,
            pltpu.SemaphoreType.DMA,
            pltpu.SemaphoreType.DMA,
            pltpu.SemaphoreType.DMA,
            pltpu.SemaphoreType.DMA,
            pltpu.SemaphoreType.DMA,
            pltpu.SemaphoreType.DMA,
        ],
    )
    def sc_kernel(x_hbm, edges_hbm,
                  s_out, xs_out,
                  sq0, sq1, dq0, dq1, rb0, rb1, rb2, rb3,
                  xb0, xb1, ones_v, dloc_v, deg_sh, acc_sh,
                  dsem, gs0, gs1, gs2, gs3, ss0, ss1, ss2, ss3):
        c = lax.axis_index("c")
        s = lax.axis_index("s")
        row0 = s * RPT
        rbufs = [rb0, rb1, rb2, rb3]
        gsems = [gs0, gs1, gs2, gs3]
        ssems = [ss0, ss1, ss2, ss3]
        sqs = [sq0, sq1]
        dqs = [dq0, dq1]
        xbufs = [xb0, xb1]
        cbase = s * BCH + jnp.maximum(s - (NSUB - NXTRA), 0)
        has_extra = s >= (NSUB - NXTRA)

        for j in range(8):
            ones_v[pl.ds(j * 16, 16)] = jnp.ones((16,), jnp.float32)

        def _zero_dloc(k, _):
            dloc_v[pl.ds(k * 16, 16)] = jnp.zeros((16,), jnp.float32)
            return 0
        lax.fori_loop(0, RPT // 16, _zero_dloc, 0)
        pltpu.sync_copy(dloc_v, deg_sh.at[pl.ds(row0, RPT)])
        plsc.subcore_barrier()

        _scB = jax.named_scope("phaseB_deg"); _scB.__enter__()
        for q in range(BCH // QCH):
            dq = dqs[q % 2]
            pltpu.sync_copy(
                edges_hbm.at[pl.ds(NROWS + cbase + q * QCH, QCH)], dq)

            def _deg(i, _, dq=dq):
                pltpu.async_copy(ones_v, deg_sh.at[dq.at[i]], dsem, add=True)

                @pl.when(i >= 8)
                def _():
                    pltpu.make_async_copy(
                        ones_v, deg_sh.at[dq.at[i - 8]], dsem).wait()
                return 0
            lax.fori_loop(0, QCH, _deg, 0)
            for i in range(QCH - 8, QCH):
                pltpu.make_async_copy(
                    ones_v, deg_sh.at[dq.at[i]], dsem).wait()

        @pl.when(has_extra)
        def _():
            pltpu.sync_copy(
                edges_hbm.at[pl.ds(NROWS + cbase + BCH, 1)],
                dqs[0].at[pl.ds(0, 1)])
            pltpu.sync_copy(ones_v, deg_sh.at[dqs[0].at[0]], add=True)
        plsc.subcore_barrier()
        _scB.__exit__(None, None, None)

        pltpu.sync_copy(deg_sh.at[pl.ds(row0, RPT)], dloc_v)

        def _dinv(k, _):
            dv = dloc_v[pl.ds(k * 16, 16)] + 1.0
            bits = lax.bitcast_convert_type(dv, jnp.int32)
            y = lax.bitcast_convert_type(
                jnp.int32(0x5F3759DF) - (bits >> 1), jnp.float32)
            half = dv * 0.5
            y = y * (1.5 - half * y * y)
            y = y * (1.5 - half * y * y)
            y = y * (1.5 - half * y * y)
            dloc_v[pl.ds(k * 16, 16)] = y
            return 0
        lax.fori_loop(0, RPT // 16, _dinv, 0)

        _scD = jax.named_scope("phaseD_scale"); _scD.__enter__()
        in_sems = [gs0, gs1]
        o0_sems = [gs2, gs3]
        o1_sems = [ss0, ss1]
        acc_sems = [ss2, ss3]

        def _d_bufs(ch):
            p = ch % 2
            return (rbufs[0].at[pl.ds(64 * p, 64)],
                    rbufs[1].at[pl.ds(64 * p, 64)])

        last = NSUB - 1

        def _d_in_start(ch):
            if ch * 64 + 64 <= N - last * RPT:
                pltpu.async_copy(
                    x_hbm.at[pl.ds(row0 + ch * 64, 64)],
                    xbufs[ch % 2], in_sems[ch % 2])
            else:
                @pl.when(s < last)
                def _():
                    pltpu.async_copy(
                        x_hbm.at[pl.ds(row0 + ch * 64, 64)],
                        xbufs[ch % 2], in_sems[ch % 2])
                tail = N - last * RPT - ch * 64
                if 0 < tail < 64:
                    @pl.when(s == last)
                    def _():
                        pltpu.async_copy(
                            x_hbm.at[pl.ds(row0 + ch * 64, 16)],
                            xbufs[ch % 2].at[pl.ds(0, 16)],
                            in_sems[ch % 2])

        def _d_in_wait(ch):
            if ch * 64 + 64 <= N - last * RPT:
                pltpu.make_async_copy(
                    x_hbm.at[pl.ds(row0 + ch * 64, 64)],
                    xbufs[ch % 2], in_sems[ch % 2]).wait()
            else:
                @pl.when(s < last)
                def _():
                    pltpu.make_async_copy(
                        x_hbm.at[pl.ds(row0 + ch * 64, 64)],
                        xbufs[ch % 2], in_sems[ch % 2]).wait()
                tail = N - last * RPT - ch * 64
                if 0 < tail < 64:
                    @pl.when(s == last)
                    def _():
                        pltpu.make_async_copy(
                            x_hbm.at[pl.ds(row0 + ch * 64, 16)],
                            xbufs[ch % 2].at[pl.ds(0, 16)],
                            in_sems[ch % 2]).wait()

        def _d_out_start(ch):
            r0 = row0 + ch * 64
            p = ch % 2
            oh0, oh1 = _d_bufs(ch)
            pltpu.async_copy(oh0, xs_out.at[0, pl.ds(r0, 64)], o0_sems[p])
            pltpu.async_copy(oh1, xs_out.at[1, pl.ds(r0, 64)], o1_sems[p])

            @pl.when(c == 0)
            def _():
                pltpu.async_copy(
                    oh0, acc_sh.at[pl.ds(r0, 64)], acc_sems[p])

            @pl.when(c == 1)
            def _():
                pltpu.async_copy(
                    oh1, acc_sh.at[pl.ds(r0, 64)], acc_sems[p])

        def _d_out_wait(ch):
            r0 = row0 + ch * 64
            p = ch % 2
            oh0, oh1 = _d_bufs(ch)
            pltpu.make_async_copy(
                oh0, xs_out.at[0, pl.ds(r0, 64)], o0_sems[p]).wait()
            pltpu.make_async_copy(
                oh1, xs_out.at[1, pl.ds(r0, 64)], o1_sems[p]).wait()

            @pl.when(c == 0)
            def _():
                pltpu.make_async_copy(
                    oh0, acc_sh.at[pl.ds(r0, 64)], acc_sems[p]).wait()

            @pl.when(c == 1)
            def _():
                pltpu.make_async_copy(
                    oh1, acc_sh.at[pl.ds(r0, 64)], acc_sems[p]).wait()

        _d_in_start(0)
        for ch in range(DCH):
            if ch + 1 < DCH:
                _d_in_start(ch + 1)
            _d_in_wait(ch)
            if ch >= 2:
                _d_out_wait(ch - 2)
            p = ch % 2
            xin = xbufs[p]
            ob0, ob1 = rbufs[0], rbufs[1]

            def _row(g, _, ch=ch, xin=xin, p=p, ob0=ob0, ob1=ob1):
                dvbs = []
                for u in range(4):
                    idxv = jnp.zeros((16,), jnp.int32) + (ch * 64 + g * 4 + u)
                    dvbs.append(plsc.load_gather(dloc_v, [idxv]))
                for u in range(4):
                    rr = g * 4 + u
                    for j in range(4):
                        sl = pl.ds(j * 16, 16)
                        ob0[64 * p + rr, sl] = xin[rr, sl] * dvbs[u]
                    for j in range(4):
                        sl = pl.ds(j * 16, 16)
                        ob1[64 * p + rr, sl] = (
                            xin[rr, pl.ds(64 + j * 16, 16)] * dvbs[u])
                return 0
            lax.fori_loop(0, 16, _row, 0)
            _d_out_start(ch)
        _d_out_wait(DCH - 2)
        _d_out_wait(DCH - 1)
        plsc.subcore_barrier()
        _scD.__exit__(None, None, None)
        _scE = jax.named_scope("phaseE_main"); _scE.__enter__()

        def _gstart(k, sq, i):
            pltpu.async_copy(
                xs_out.at[c].at[sq.at[i]], rbufs[k], gsems[k])

        def _gwait(k, sq, i):
            pltpu.make_async_copy(
                xs_out.at[c].at[sq.at[i]], rbufs[k], gsems[k]).wait()

        def _sstart(k, dq, i):
            pltpu.async_copy(
                rbufs[k], acc_sh.at[dq.at[i]], ssems[k], add=True)

        def _swait(k, dq, i):
            pltpu.make_async_copy(
                rbufs[k], acc_sh.at[dq.at[i]], ssems[k]).wait()

        for q in range(BCH // QCH):
            sq = sqs[q % 2]
            dq = dqs[q % 2]
            pltpu.sync_copy(
                edges_hbm.at[pl.ds(cbase + q * QCH, QCH)], sq)
            pltpu.sync_copy(
                edges_hbm.at[pl.ds(NROWS + cbase + q * QCH, QCH)], dq)
            for k in range(NBUF):
                _gstart(k, sq, k)

            def _quad(p, _, sq=sq, dq=dq):
                i0 = NBUF * p
                for k in range(NBUF):
                    i = i0 + k
                    _gwait(k, sq, i)
                    _sstart(k, dq, i)
                    km1 = (k - 1) % NBUF
                    if km1 == NBUF - 1:
                        @pl.when(p > 0)
                        def _():
                            _swait(km1, dq, i - 1)
                            _gstart(km1, sq, i + NBUF - 1)
                    else:
                        @pl.when(i + NBUF - 1 < QCH)
                        def _():
                            _swait(km1, dq, i - 1)
                            _gstart(km1, sq, i + NBUF - 1)
                return 0
            lax.fori_loop(0, QCH // NBUF, _quad, 0)
            for k in range(NBUF - 1):
                _swait(k, dq, QCH - NBUF + k)
            _swait(NBUF - 1, dq, QCH - 1)

        @pl.when(has_extra)
        def _():
            pltpu.sync_copy(
                edges_hbm.at[pl.ds(cbase + BCH, 1)], sqs[0].at[pl.ds(0, 1)])
            pltpu.sync_copy(
                edges_hbm.at[pl.ds(NROWS + cbase + BCH, 1)],
                dqs[0].at[pl.ds(0, 1)])
            pltpu.async_copy(
                xs_out.at[c].at[sqs[0].at[0]], rbufs[0], gsems[0])
            pltpu.make_async_copy(
                xs_out.at[c].at[sqs[0].at[0]], rbufs[0], gsems[0]).wait()
            pltpu.sync_copy(rbufs[0], acc_sh.at[dqs[0].at[0]], add=True)
        plsc.subcore_barrier()
        _scE.__exit__(None, None, None)

        def _f_in(k):
            pltpu.sync_copy(
                acc_sh.at[pl.ds(row0 + k * CHUNK, CHUNK)], rbufs[k % 2])

        def _f_out_start(k):
            pltpu.async_copy(
                rbufs[k % 2],
                s_out.at[c, pl.ds(row0 + k * CHUNK, CHUNK)], gsems[k % 2])

        def _f_out_wait(k):
            pltpu.make_async_copy(
                rbufs[k % 2],
                s_out.at[c, pl.ds(row0 + k * CHUNK, CHUNK)],
                gsems[k % 2]).wait()

        for k in range(RPT // CHUNK):
            if k >= 2:
                _f_out_wait(k - 2)
            _f_in(k)
            rbk = rbufs[k % 2]

            def _frow(g, _, k=k, rbk=rbk):
                dvbs = []
                for u in range(4):
                    idxv = jnp.zeros((16,), jnp.int32) + (k * CHUNK + g * 4 + u)
                    dvbs.append(plsc.load_gather(dloc_v, [idxv]))
                for u in range(4):
                    rr = g * 4 + u
                    for j in range(4):
                        sl = pl.ds(j * 16, 16)
                        rbk[rr, sl] = rbk[rr, sl] * dvbs[u]
                return 0
            lax.fori_loop(0, CHUNK // 4, _frow, 0)
            _f_out_start(k)
        _f_out_wait(RPT // CHUNK - 2)
        _f_out_wait(RPT // CHUNK - 1)

    return sc_kernel


_SC_KERNEL = _sc_kernel_fn()

_TC_ROWS = 2000


def _tc_body(s_ref, w_ref, b_ref, o_ref):
    u = jnp.concatenate([s_ref[0], s_ref[1]], axis=1)
    acc = jnp.dot(u, w_ref[...], preferred_element_type=jnp.float32)
    o_ref[...] = acc + b_ref[...]


def kernel(x, edge_index, W, b):
    edges = edge_index.astype(jnp.int32).reshape(2 * NROWS, CHUNK)

    S, _ = _SC_KERNEL(x, edges)

    out = pl.pallas_call(
        _tc_body,
        grid=(N // _TC_ROWS,),
        in_specs=[
            pl.BlockSpec((NCORE, _TC_ROWS, DH), lambda i: (0, i, 0)),
            pl.BlockSpec((D, D), lambda i: (0, 0)),
            pl.BlockSpec((1, D), lambda i: (0, 0)),
        ],
        out_specs=pl.BlockSpec((_TC_ROWS, D), lambda i: (i, 0)),
        out_shape=jax.ShapeDtypeStruct((N, D), jnp.float32),
    )(S, W, b.reshape(1, D))

    return out


# speedup vs baseline: 1.5088x; 1.0011x over previous
import functools

import jax
import jax.numpy as jnp
from jax import lax
from jax.experimental import pallas as pl
from jax.experimental.pallas import tpu as pltpu
from jax.experimental.pallas import tpu_sc as plsc

N = 10000
D = 128
DH = 64
E = 320000
NPAD = 10240
NCORE = 2
NSUB = 16
RPT = NPAD // NSUB
CHUNK = 128
NROWS = E // CHUNK
BCH = 156
NXTRA = NROWS - NSUB * BCH
QCH = 52
NBUF = 4
DCH = RPT // 64


def _sc_kernel_fn():
    mesh = plsc.VectorSubcoreMesh(core_axis_name="c", subcore_axis_name="s")

    @functools.partial(
        pl.kernel,
        mesh=mesh,
        compiler_params=pltpu.CompilerParams(
            use_tc_tiling_on_sc=False, needs_layout_passes=False),
        out_type=(
            jax.ShapeDtypeStruct((NCORE, NPAD, DH), jnp.float32),
            jax.ShapeDtypeStruct((NCORE, NPAD, DH), jnp.float32),
        ),
        scratch_types=[
            pltpu.VMEM((QCH, CHUNK), jnp.int32),
            pltpu.VMEM((QCH, CHUNK), jnp.int32),
            pltpu.VMEM((QCH, CHUNK), jnp.int32),
            pltpu.VMEM((QCH, CHUNK), jnp.int32),
            pltpu.VMEM((CHUNK, DH), jnp.float32),
            pltpu.VMEM((CHUNK, DH), jnp.float32),
            pltpu.VMEM((CHUNK, DH), jnp.float32),
            pltpu.VMEM((CHUNK, DH), jnp.float32),
            pltpu.VMEM((64, D), jnp.float32),
            pltpu.VMEM((64, D), jnp.float32),
            pltpu.VMEM((CHUNK,), jnp.float32),
            pltpu.VMEM((RPT,), jnp.float32),
            pltpu.VMEM_SHARED((NPAD,), jnp.float32),
            pltpu.VMEM_SHARED((NPAD, DH), jnp.float32),
            pltpu.SemaphoreType.DMA,
            pltpu.SemaphoreType.DMA,
            pltpu.SemaphoreType.DMA,
            pltpu.SemaphoreType.DMA,
            pltpu.SemaphoreType.DMA,
            pltpu.SemaphoreType.DMA,
            pltpu.SemaphoreType.DMA,
            pltpu.SemaphoreType.DMA,
            pltpu.SemaphoreType.DMA,
        ],
    )
    def sc_kernel(x_hbm, edges_hbm,
                  s_out, xs_out,
                  sq0, sq1, dq0, dq1, rb0, rb1, rb2, rb3,
                  xb0, xb1, ones_v, dloc_v, deg_sh, acc_sh,
                  dsem, gs0, gs1, gs2, gs3, ss0, ss1, ss2, ss3):
        c = lax.axis_index("c")
        s = lax.axis_index("s")
        row0 = s * RPT
        rbufs = [rb0, rb1, rb2, rb3]
        gsems = [gs0, gs1, gs2, gs3]
        ssems = [ss0, ss1, ss2, ss3]
        sqs = [sq0, sq1]
        dqs = [dq0, dq1]
        xbufs = [xb0, xb1]
        cbase = s * BCH + jnp.maximum(s - (NSUB - NXTRA), 0)
        has_extra = s >= (NSUB - NXTRA)

        for j in range(8):
            ones_v[pl.ds(j * 16, 16)] = jnp.ones((16,), jnp.float32)

        def _zero_dloc(k, _):
            dloc_v[pl.ds(k * 16, 16)] = jnp.zeros((16,), jnp.float32)
            return 0
        lax.fori_loop(0, RPT // 16, _zero_dloc, 0)
        pltpu.sync_copy(dloc_v, deg_sh.at[pl.ds(row0, RPT)])
        plsc.subcore_barrier()

        for q in range(BCH // QCH):
            dq = dqs[q % 2]
            pltpu.sync_copy(
                edges_hbm.at[pl.ds(NROWS + cbase + q * QCH, QCH)], dq)

            def _deg(i, _, dq=dq):
                pltpu.async_copy(ones_v, deg_sh.at[dq.at[i]], dsem, add=True)

                @pl.when(i >= 8)
                def _():
                    pltpu.make_async_copy(
                        ones_v, deg_sh.at[dq.at[i - 8]], dsem).wait()
                return 0
            lax.fori_loop(0, QCH, _deg, 0)
            for i in range(QCH - 8, QCH):
                pltpu.make_async_copy(
                    ones_v, deg_sh.at[dq.at[i]], dsem).wait()

        @pl.when(has_extra)
        def _():
            pltpu.sync_copy(
                edges_hbm.at[pl.ds(NROWS + cbase + BCH, 1)],
                dqs[0].at[pl.ds(0, 1)])
            pltpu.sync_copy(ones_v, deg_sh.at[dqs[0].at[0]], add=True)
        plsc.subcore_barrier()

        pltpu.sync_copy(deg_sh.at[pl.ds(row0, RPT)], dloc_v)

        def _dinv(k, _):
            dv = dloc_v[pl.ds(k * 16, 16)] + 1.0
            bits = lax.bitcast_convert_type(dv, jnp.int32)
            y = lax.bitcast_convert_type(
                jnp.int32(0x5F3759DF) - (bits >> 1), jnp.float32)
            half = dv * 0.5
            y = y * (1.5 - half * y * y)
            y = y * (1.5 - half * y * y)
            y = y * (1.5 - half * y * y)
            dloc_v[pl.ds(k * 16, 16)] = y
            return 0
        lax.fori_loop(0, RPT // 16, _dinv, 0)

        in_sems = [gs0, gs1]
        o0_sems = [gs2, gs3]
        o1_sems = [ss0, ss1]
        acc_sems = [ss2, ss3]

        def _d_bufs(ch):
            p = ch % 2
            return (rbufs[0].at[pl.ds(64 * p, 64)],
                    rbufs[1].at[pl.ds(64 * p, 64)])

        last = NSUB - 1

        def _d_in_start(ch):
            if ch * 64 + 64 <= N - last * RPT:
                pltpu.async_copy(
                    x_hbm.at[pl.ds(row0 + ch * 64, 64)],
                    xbufs[ch % 2], in_sems[ch % 2])
            else:
                @pl.when(s < last)
                def _():
                    pltpu.async_copy(
                        x_hbm.at[pl.ds(row0 + ch * 64, 64)],
                        xbufs[ch % 2], in_sems[ch % 2])
                tail = N - last * RPT - ch * 64
                if 0 < tail < 64:
                    @pl.when(s == last)
                    def _():
                        pltpu.async_copy(
                            x_hbm.at[pl.ds(row0 + ch * 64, 16)],
                            xbufs[ch % 2].at[pl.ds(0, 16)],
                            in_sems[ch % 2])

        def _d_in_wait(ch):
            if ch * 64 + 64 <= N - last * RPT:
                pltpu.make_async_copy(
                    x_hbm.at[pl.ds(row0 + ch * 64, 64)],
                    xbufs[ch % 2], in_sems[ch % 2]).wait()
            else:
                @pl.when(s < last)
                def _():
                    pltpu.make_async_copy(
                        x_hbm.at[pl.ds(row0 + ch * 64, 64)],
                        xbufs[ch % 2], in_sems[ch % 2]).wait()
                tail = N - last * RPT - ch * 64
                if 0 < tail < 64:
                    @pl.when(s == last)
                    def _():
                        pltpu.make_async_copy(
                            x_hbm.at[pl.ds(row0 + ch * 64, 16)],
                            xbufs[ch % 2].at[pl.ds(0, 16)],
                            in_sems[ch % 2]).wait()

        def _d_out_start(ch):
            r0 = row0 + ch * 64
            p = ch % 2
            oh0, oh1 = _d_bufs(ch)
            pltpu.async_copy(oh0, xs_out.at[0, pl.ds(r0, 64)], o0_sems[p])
            pltpu.async_copy(oh1, xs_out.at[1, pl.ds(r0, 64)], o1_sems[p])

            @pl.when(c == 0)
            def _():
                pltpu.async_copy(
                    oh0, acc_sh.at[pl.ds(r0, 64)], acc_sems[p])

            @pl.when(c == 1)
            def _():
                pltpu.async_copy(
                    oh1, acc_sh.at[pl.ds(r0, 64)], acc_sems[p])

        def _d_out_wait(ch):
            r0 = row0 + ch * 64
            p = ch % 2
            oh0, oh1 = _d_bufs(ch)
            pltpu.make_async_copy(
                oh0, xs_out.at[0, pl.ds(r0, 64)], o0_sems[p]).wait()
            pltpu.make_async_copy(
                oh1, xs_out.at[1, pl.ds(r0, 64)], o1_sems[p]).wait()

            @pl.when(c == 0)
            def _():
                pltpu.make_async_copy(
                    oh0, acc_sh.at[pl.ds(r0, 64)], acc_sems[p]).wait()

            @pl.when(c == 1)
            def _():
                pltpu.make_async_copy(
                    oh1, acc_sh.at[pl.ds(r0, 64)], acc_sems[p]).wait()

        _d_in_start(0)
        for ch in range(DCH):
            if ch + 1 < DCH:
                _d_in_start(ch + 1)
            _d_in_wait(ch)
            if ch >= 2:
                _d_out_wait(ch - 2)
            p = ch % 2
            xin = xbufs[p]
            ob0, ob1 = rbufs[0], rbufs[1]

            def _row(g, _, ch=ch, xin=xin, p=p, ob0=ob0, ob1=ob1):
                dvbs = []
                for u in range(4):
                    idxv = jnp.zeros((16,), jnp.int32) + (ch * 64 + g * 4 + u)
                    dvbs.append(plsc.load_gather(dloc_v, [idxv]))
                for u in range(4):
                    rr = g * 4 + u
                    for j in range(4):
                        sl = pl.ds(j * 16, 16)
                        ob0[64 * p + rr, sl] = xin[rr, sl] * dvbs[u]
                    for j in range(4):
                        sl = pl.ds(j * 16, 16)
                        ob1[64 * p + rr, sl] = (
                            xin[rr, pl.ds(64 + j * 16, 16)] * dvbs[u])
                return 0
            lax.fori_loop(0, 16, _row, 0)
            _d_out_start(ch)
        _d_out_wait(DCH - 2)
        _d_out_wait(DCH - 1)
        plsc.subcore_barrier()

        def _gstart(k, sq, i):
            pltpu.async_copy(
                xs_out.at[c].at[sq.at[i]], rbufs[k], gsems[k])

        def _gwait(k, sq, i):
            pltpu.make_async_copy(
                xs_out.at[c].at[sq.at[i]], rbufs[k], gsems[k]).wait()

        def _sstart(k, dq, i):
            pltpu.async_copy(
                rbufs[k], acc_sh.at[dq.at[i]], ssems[k], add=True)

        def _swait(k, dq, i):
            pltpu.make_async_copy(
                rbufs[k], acc_sh.at[dq.at[i]], ssems[k]).wait()

        for q in range(BCH // QCH):
            sq = sqs[q % 2]
            dq = dqs[q % 2]
            pltpu.sync_copy(
                edges_hbm.at[pl.ds(cbase + q * QCH, QCH)], sq)
            pltpu.sync_copy(
                edges_hbm.at[pl.ds(NROWS + cbase + q * QCH, QCH)], dq)
            for k in range(NBUF):
                _gstart(k, sq, k)

            def _quad(p, _, sq=sq, dq=dq):
                i0 = NBUF * p
                for k in range(NBUF):
                    i = i0 + k
                    _gwait(k, sq, i)
                    _sstart(k, dq, i)
                    km1 = (k - 1) % NBUF
                    if km1 == NBUF - 1:
                        @pl.when(p > 0)
                        def _():
                            _swait(km1, dq, i - 1)
                            _gstart(km1, sq, i + NBUF - 1)
                    else:
                        @pl.when(i + NBUF - 1 < QCH)
                        def _():
                            _swait(km1, dq, i - 1)
                            _gstart(km1, sq, i + NBUF - 1)
                return 0
            lax.fori_loop(0, QCH // NBUF, _quad, 0)
            for k in range(NBUF - 1):
                _swait(k, dq, QCH - NBUF + k)
            _swait(NBUF - 1, dq, QCH - 1)

        @pl.when(has_extra)
        def _():
            pltpu.sync_copy(
                edges_hbm.at[pl.ds(cbase + BCH, 1)], sqs[0].at[pl.ds(0, 1)])
            pltpu.sync_copy(
                edges_hbm.at[pl.ds(NROWS + cbase + BCH, 1)],
                dqs[0].at[pl.ds(0, 1)])
            pltpu.async_copy(
                xs_out.at[c].at[sqs[0].at[0]], rbufs[0], gsems[0])
            pltpu.make_async_copy(
                xs_out.at[c].at[sqs[0].at[0]], rbufs[0], gsems[0]).wait()
            pltpu.sync_copy(rbufs[0], acc_sh.at[dqs[0].at[0]], add=True)
        plsc.subcore_barrier()

        def _f_in(k):
            pltpu.sync_copy(
                acc_sh.at[pl.ds(row0 + k * CHUNK, CHUNK)], rbufs[k % 2])

        def _f_out_start(k):
            pltpu.async_copy(
                rbufs[k % 2],
                s_out.at[c, pl.ds(row0 + k * CHUNK, CHUNK)], gsems[k % 2])

        def _f_out_wait(k):
            pltpu.make_async_copy(
                rbufs[k % 2],
                s_out.at[c, pl.ds(row0 + k * CHUNK, CHUNK)],
                gsems[k % 2]).wait()

        for k in range(RPT // CHUNK):
            if k >= 2:
                _f_out_wait(k - 2)
            _f_in(k)
            rbk = rbufs[k % 2]

            def _frow(g, _, k=k, rbk=rbk):
                dvbs = []
                for u in range(4):
                    idxv = jnp.zeros((16,), jnp.int32) + (k * CHUNK + g * 4 + u)
                    dvbs.append(plsc.load_gather(dloc_v, [idxv]))
                for u in range(4):
                    rr = g * 4 + u
                    for j in range(4):
                        sl = pl.ds(j * 16, 16)
                        rbk[rr, sl] = rbk[rr, sl] * dvbs[u]
                return 0
            lax.fori_loop(0, CHUNK // 4, _frow, 0)
            _f_out_start(k)
        _f_out_wait(RPT // CHUNK - 2)
        _f_out_wait(RPT // CHUNK - 1)

    return sc_kernel


_SC_KERNEL = _sc_kernel_fn()

_TC_ROWS = 2000


def _tc_body(s_ref, w_ref, b_ref, o_ref):
    u = jnp.concatenate([s_ref[0], s_ref[1]], axis=1)
    acc = jnp.dot(u, w_ref[...], preferred_element_type=jnp.float32)
    o_ref[...] = acc + b_ref[...]


def kernel(x, edge_index, W, b):
    edges = edge_index.astype(jnp.int32).reshape(2 * NROWS, CHUNK)

    S, _ = _SC_KERNEL(x, edges)

    out = pl.pallas_call(
        _tc_body,
        grid=(N // _TC_ROWS,),
        in_specs=[
            pl.BlockSpec((NCORE, _TC_ROWS, DH), lambda i: (0, i, 0)),
            pl.BlockSpec((D, D), lambda i: (0, 0)),
            pl.BlockSpec((1, D), lambda i: (0, 0)),
        ],
        out_specs=pl.BlockSpec((_TC_ROWS, D), lambda i: (i, 0)),
        out_shape=jax.ShapeDtypeStruct((N, D), jnp.float32),
    )(S, W, b.reshape(1, D))

    return out
